# baseline (speedup 1.0000x reference)
import functools

import jax
import jax.numpy as jnp
from jax import lax
from jax.experimental import pallas as pl
from jax.experimental.pallas import tpu as pltpu
from jax.experimental.pallas import tpu_sc as plsc

NN = 10000
EE = 160000
GG = 64
NP = 10240
EP = 163840
NW = 32
EPW = EP // NW
NPW = NP // NW
RB = 512
NB = NP // RB
F32 = jnp.float32
I32 = jnp.int32

_SC_PARAMS = pltpu.CompilerParams(use_tc_tiling_on_sc=False,
                                  needs_layout_passes=False)


def _fs(shape):
    return pl.BlockSpec(shape, lambda i: tuple(0 for _ in shape))


def _rows_mask(i):
    rows = lax.broadcasted_iota(I32, (RB, 1), 0) + i * RB
    return rows < NN


def _dot(a, b, hi=False):
    return jnp.dot(a, b, preferred_element_type=F32,
                   precision=(lax.Precision.HIGHEST if hi
                              else lax.Precision.DEFAULT))


def _embed_nodes(xi_p, pos_p, e1, e2, e3, e4, w1top):
    def body(xi_ref, pos_ref, e1_ref, e2_ref, e3_ref, e4_ref, wt_ref,
             x_ref, a_ref):
        xi = xi_ref[...]
        oh1 = (xi[:, 0:1] == lax.broadcasted_iota(I32, (RB, 16), 1)).astype(F32)
        oh2 = (xi[:, 1:2] == lax.broadcasted_iota(I32, (RB, 8), 1)).astype(F32)
        oh3 = (xi[:, 2:3] == lax.broadcasted_iota(I32, (RB, 8), 1)).astype(F32)
        oh4 = (xi[:, 3:4] == lax.broadcasted_iota(I32, (RB, 8), 1)).astype(F32)
        c1 = _dot(oh1, e1_ref[...], hi=True)
        c2 = (_dot(oh2, e2_ref[...], hi=True)
              + _dot(oh3, e3_ref[...], hi=True)
              + _dot(oh4, e4_ref[...], hi=True))
        c3 = (xi[:, 4:5] - 1).astype(F32)
        x_ref[...] = jnp.concatenate(
            [c1, c2, c3, jnp.zeros((RB, 15), F32)], axis=1)
        a_ref[...] = _dot(pos_ref[...], wt_ref[...])

    return pl.pallas_call(
        body, grid=(NB,),
        in_specs=[pl.BlockSpec((RB, 8), lambda i: (i, 0)),
                  pl.BlockSpec((RB, 16), lambda i: (i, 0)),
                  _fs((16, 32)), _fs((8, 32)), _fs((8, 32)), _fs((8, 32)),
                  _fs((16, 64))],
        out_specs=[pl.BlockSpec((RB, 80), lambda i: (i, 0)),
                   pl.BlockSpec((RB, 64), lambda i: (i, 0))],
        out_shape=[jax.ShapeDtypeStruct((NP, 80), F32),
                   jax.ShapeDtypeStruct((NP, 64), F32)],
    )(xi_p, pos_p, e1, e2, e3, e4, w1top)


def _embed_edges(ai_p, embp, w1, bb1, w2, bb2, w3, bb3, w4, bb4):
    def body(ai_ref, emb_ref, w1_ref, b1_ref, w2_ref, b2_ref, w3_ref, b3_ref,
             w4_ref, b4_ref, o1_ref, o2_ref, o3_ref, o4_ref):
        ai = ai_ref[...]
        oh = (ai[:, 0:1] == lax.broadcasted_iota(I32, (RB, 8), 1)).astype(F32)
        et = _dot(oh, emb_ref[...], hi=True)
        ea = jnp.concatenate([et[:, :15], (ai[:, 1:2] - 1).astype(F32)], axis=1)
        o1_ref[...] = _dot(ea, w1_ref[...]) + b1_ref[...]
        o2_ref[...] = _dot(ea, w2_ref[...]) + b2_ref[...]
        o3_ref[...] = _dot(ea, w3_ref[...]) + b3_ref[...]
        o4_ref[...] = _dot(ea, w4_ref[...]) + b4_ref[...]

    return pl.pallas_call(
        body, grid=(EP // RB,),
        in_specs=[pl.BlockSpec((RB, 8), lambda i: (i, 0)),
                  _fs((8, 16)), _fs((16, 80)), _fs((1, 80)),
                  _fs((16, 32)), _fs((1, 32)), _fs((16, 32)), _fs((1, 32)),
                  _fs((16, 32)), _fs((1, 32))],
        out_specs=[pl.BlockSpec((RB, 80), lambda i: (i, 0)),
                   pl.BlockSpec((RB, 32), lambda i: (i, 0)),
                   pl.BlockSpec((RB, 32), lambda i: (i, 0)),
                   pl.BlockSpec((RB, 32), lambda i: (i, 0))],
        out_shape=[jax.ShapeDtypeStruct((EP, 80), F32),
                   jax.ShapeDtypeStruct((EP, 32), F32),
                   jax.ShapeDtypeStruct((EP, 32), F32),
                   jax.ShapeDtypeStruct((EP, 32), F32)],
    )(ai_p, embp, w1, bb1, w2, bb2, w3, bb3, w4, bb4)


def _gine_mlp(x, parts, wa, ba, wb, bb, eps, F, Fh, Fo):
    def body(x_ref, p_ref, wa_ref, ba_ref, wb_ref, bb_ref, eps_ref, y_ref):
        h = (1.0 + eps_ref[0, 0]) * x_ref[...] + p_ref[0] + p_ref[1]
        t = jnp.maximum(_dot(h, wa_ref[...]) + ba_ref[...], 0.0)
        y_ref[...] = jnp.maximum(_dot(t, wb_ref[...]) + bb_ref[...], 0.0)

    return pl.pallas_call(
        body, grid=(NB,),
        in_specs=[pl.BlockSpec((RB, F), lambda i: (i, 0)),
                  pl.BlockSpec((2, RB, F), lambda i: (0, i, 0)),
                  _fs((F, Fh)), _fs((1, Fh)), _fs((Fh, Fo)), _fs((1, Fo)),
                  _fs((1, 1))],
        out_specs=pl.BlockSpec((RB, Fo), lambda i: (i, 0)),
        out_shape=jax.ShapeDtypeStruct((NP, Fo), F32),
    )(x, parts, wa, ba, wb, bb, eps)


def _stats2(y, F, relu_in=False):
    def body(y_ref, st_ref):
        i = pl.program_id(0)
        yv = y_ref[...]
        if relu_in:
            yv = jnp.maximum(yv, 0.0)
        mask = _rows_mask(i % NB)

        @pl.when(i == 0)
        def _():
            st_ref[...] = jnp.zeros((8, F), F32)

        @pl.when(i < NB)
        def _():
            ym = jnp.where(mask, yv, 0.0)
            st_ref[0:1, :] += jnp.sum(ym, axis=0, keepdims=True)

        @pl.when(i >= NB)
        def _():
            mean = st_ref[0:1, :] / NN
            d = jnp.where(mask, yv - mean, 0.0)
            st_ref[1:2, :] += jnp.sum(d * d, axis=0, keepdims=True)

    return pl.pallas_call(
        body, grid=(2 * NB,),
        in_specs=[pl.BlockSpec((RB, F), lambda i: (i % NB, 0))],
        out_specs=pl.BlockSpec((8, F), lambda i: (0, 0)),
        out_shape=jax.ShapeDtypeStruct((8, F), F32),
    )(y)


def _bn_apply(y, st, g2, F, relu_in=False, proj=None):
    H = proj.shape[1] if proj is not None else 0

    def body(*refs):
        if proj is not None:
            y_ref, st_ref, g_ref, wt_ref, xn_ref, a_ref = refs
        else:
            y_ref, st_ref, g_ref, xn_ref = refs
        mean = st_ref[0:1, :] / NN
        var = st_ref[1:2, :] / NN
        rs = lax.rsqrt(var + 1e-5)
        yv = y_ref[...]
        if relu_in:
            yv = jnp.maximum(yv, 0.0)
        xn = (yv - mean) * rs * g_ref[0:1, :] + g_ref[1:2, :]
        xn_ref[...] = xn
        if proj is not None:
            a_ref[...] = _dot(xn, wt_ref[...])

    in_specs = [pl.BlockSpec((RB, F), lambda i: (i, 0)),
                _fs((8, F)), _fs((2, F))]
    out_specs = [pl.BlockSpec((RB, F), lambda i: (i, 0))]
    out_shape = [jax.ShapeDtypeStruct((NP, F), F32)]
    args = [y, st, g2]
    if proj is not None:
        in_specs += [_fs((F, H))]
        out_specs += [pl.BlockSpec((RB, H), lambda i: (i, 0))]
        out_shape += [jax.ShapeDtypeStruct((NP, H), F32)]
        args += [proj]
    res = pl.pallas_call(body, grid=(NB,), in_specs=in_specs,
                         out_specs=out_specs, out_shape=out_shape)(*args)
    return res if proj is not None else res[0]


def _edge_mlp(D, Ag, wbot, b1, wb, b2, Fd, H):
    def body(d_ref, ag_ref, wbot_ref, b1_ref, wb_ref, b2_ref, m_ref):
        rh = jnp.maximum(
            ag_ref[...] + _dot(d_ref[...], wbot_ref[...]) + b1_ref[...], 0.0)
        m_ref[...] = _dot(rh, wb_ref[...]) + b2_ref[...]

    return pl.pallas_call(
        body, grid=(EP // RB,),
        in_specs=[pl.BlockSpec((RB, Fd), lambda i: (i, 0)),
                  pl.BlockSpec((RB, H), lambda i: (i, 0)),
                  _fs((Fd, H)), _fs((1, H)), _fs((H, H)), _fs((1, H))],
        out_specs=pl.BlockSpec((RB, H), lambda i: (i, 0)),
        out_shape=jax.ShapeDtypeStruct((EP, H), F32),
    )(D, Ag, wbot, b1, wb, b2)


def _pool_final(xe4, x4, batch_p, g2):
    def body(xe_ref, x4_ref, b_ref, g_ref, o_ref, acc):
        i = pl.program_id(0)

        @pl.when(i == 0)
        def _():
            acc[...] = jnp.zeros((GG, 256), F32)

        oh = (b_ref[...] == lax.broadcasted_iota(I32, (RB, GG), 1)).astype(F32)
        xcat = jnp.concatenate([xe_ref[...], x4_ref[...]], axis=1)
        acc[...] += lax.dot_general(oh, xcat, (((0,), (0,)), ((), ())),
                                    preferred_element_type=F32,
                                    precision=lax.Precision.HIGHEST)

        @pl.when(i == NB - 1)
        def _():
            pv = jnp.maximum(acc[...], 0.0)
            m = jnp.sum(pv, axis=0, keepdims=True) / GG
            d = pv - m
            v = jnp.sum(d * d, axis=0, keepdims=True) / GG
            rs = lax.rsqrt(v + 1e-5)
            o_ref[...] = d * rs * g_ref[0:1, :] + g_ref[1:2, :]

    return pl.pallas_call(
        body, grid=(NB,),
        in_specs=[pl.BlockSpec((RB, 208), lambda i: (i, 0)),
                  pl.BlockSpec((RB, 48), lambda i: (i, 0)),
                  pl.BlockSpec((RB, 1), lambda i: (i, 0)),
                  _fs((2, 256))],
        out_specs=pl.BlockSpec((GG, 256), lambda i: (0, 0)),
        out_shape=jax.ShapeDtypeStruct((GG, 256), F32),
        scratch_shapes=[pltpu.VMEM((GG, 256), F32)],
    )(xe4, x4, batch_p, g2)


@functools.cache
def _mesh():
    return plsc.VectorSubcoreMesh(core_axis_name="c", subcore_axis_name="s",
                                  num_cores=2, num_subcores=16)


def _gine_agg(x, eam, srcp, dstp, F):
    C = 128
    NSUB = 16
    ZR = NP // NSUB

    @functools.partial(
        pl.kernel,
        out_type=jax.ShapeDtypeStruct((2, NP, F), F32),
        mesh=_mesh(),
        compiler_params=_SC_PARAMS,
        scratch_types=[
            pltpu.VMEM((C,), I32),
            pltpu.VMEM((C,), I32),
            pltpu.VMEM((C, F), F32),
            pltpu.VMEM((C, F), F32),
            pltpu.VMEM((C, F), F32),
            pltpu.VMEM_SHARED((NP, F), F32),
            pltpu.SemaphoreType.DMA,
        ])
    def k(x_h, eam_h, src_h, dst_h, out_h, src_v, dst_v, rows_v, eam_v,
          msg_v, acc, sem):
        c = lax.axis_index("c")
        s = lax.axis_index("s")
        wid = s * 2 + c

        def zr_row(i, _):
            def zc(j, _):
                msg_v[i, pl.ds(j * 16, 16)] = jnp.zeros((16,), F32)
                return _
            return lax.fori_loop(0, F // 16, zc, _)
        lax.fori_loop(0, C, zr_row, None)

        def zcopy(i, _):
            pltpu.sync_copy(msg_v, acc.at[pl.ds(s * ZR + i * C, C)])
            return _
        lax.fori_loop(0, ZR // C, zcopy, None)
        plsc.subcore_barrier()

        ebase = wid * EPW

        def chunk(i, _):
            b = ebase + i * C
            pltpu.sync_copy(src_h.at[pl.ds(b, C)], src_v)
            pltpu.sync_copy(dst_h.at[pl.ds(b, C)], dst_v)
            pltpu.async_copy(x_h.at[src_v], rows_v, sem).wait()
            pltpu.sync_copy(eam_h.at[pl.ds(b, C), :], eam_v)

            def erow(e, _):
                def ecol(j, _):
                    sl = pl.ds(j * 16, 16)
                    msg_v[e, sl] = jnp.maximum(rows_v[e, sl] + eam_v[e, sl],
                                               0.0)
                    return _
                return lax.fori_loop(0, F // 16, ecol, _)
            lax.fori_loop(0, C, erow, None)
            pltpu.sync_copy(msg_v, acc.at[dst_v], add=True)
            return _
        lax.fori_loop(0, EPW // C, chunk, None)
        plsc.subcore_barrier()

        def wb(i, _):
            r = s * ZR + i * C
            pltpu.sync_copy(acc.at[pl.ds(r, C)], msg_v)
            pltpu.sync_copy(msg_v, out_h.at[c, pl.ds(r, C)])
            return _
        lax.fori_loop(0, ZR // C, wb, None)

    return k(x, eam, srcp, dstp)


def _edge_fetch(x, A, de_s, se_s, F, H):
    C = 64

    @functools.partial(
        pl.kernel,
        out_type=(jax.ShapeDtypeStruct((EP, F), F32),
                  jax.ShapeDtypeStruct((EP, H), F32)),
        mesh=_mesh(),
        compiler_params=_SC_PARAMS,
        scratch_types=[
            pltpu.VMEM((C,), I32),
            pltpu.VMEM((C,), I32),
            pltpu.VMEM((C, F), F32),
            pltpu.VMEM((C, F), F32),
            pltpu.VMEM((C, H), F32),
            pltpu.SemaphoreType.DMA,
            pltpu.SemaphoreType.DMA,
            pltpu.SemaphoreType.DMA,
        ])
    def k(x_h, a_h, de_h, se_h, d_out, ag_out, di_v, si_v, xs_v, xd_v, ag_v,
          sem1, sem2, sem3):
        c = lax.axis_index("c")
        s = lax.axis_index("s")
        wid = s * 2 + c
        ebase = wid * EPW

        def chunk(i, _):
            b = ebase + i * C
            pltpu.sync_copy(de_h.at[pl.ds(b, C)], di_v)
            pltpu.sync_copy(se_h.at[pl.ds(b, C)], si_v)
            c1 = pltpu.async_copy(x_h.at[si_v], xs_v, sem1)
            c2 = pltpu.async_copy(x_h.at[di_v], xd_v, sem2)
            c3 = pltpu.async_copy(a_h.at[di_v], ag_v, sem3)
            c1.wait()
            c2.wait()
            c3.wait()

            def erow(e, _):
                def ecol(j, _):
                    sl = pl.ds(j * 16, 16)
                    xs_v[e, sl] = xs_v[e, sl] - xd_v[e, sl]
                    return _
                return lax.fori_loop(0, F // 16, ecol, _)
            lax.fori_loop(0, C, erow, None)
            pltpu.sync_copy(xs_v, d_out.at[pl.ds(b, C), :])
            pltpu.sync_copy(ag_v, ag_out.at[pl.ds(b, C), :])
            return _
        lax.fori_loop(0, EPW // C, chunk, None)

    return k(x, A, de_s, se_s)


def _seg_max(m, dstp, es, H):
    C = 64

    @functools.partial(
        pl.kernel,
        out_type=jax.ShapeDtypeStruct((NP, H), F32),
        mesh=_mesh(),
        compiler_params=_SC_PARAMS,
        scratch_types=[
            pltpu.VMEM((C,), I32),
            pltpu.VMEM((C, H), F32),
            pltpu.VMEM((NPW, H), F32),
            pltpu.VMEM((48,), I32),
        ])
    def k(m_h, dst_h, es_h, out_h, di_v, m_v, acc_v, es_v):
        c = lax.axis_index("c")
        s = lax.axis_index("s")
        wid = s * 2 + c
        n0 = wid * NPW
        pltpu.sync_copy(es_h, es_v)

        def rd(idx):
            g = plsc.load_gather(es_v, [jnp.full((16,), idx, I32)])
            return jnp.max(g, axis=0)

        e0 = rd(wid)
        e1 = rd(wid + 1)
        a0 = jnp.bitwise_and(e0, jnp.int32(-C))
        nch = (e1 - a0 + (C - 1)) // C

        neg = jnp.full((16,), -jnp.inf, F32)

        def ib(i, _):
            for j in range(H // 16):
                acc_v[i, pl.ds(j * 16, 16)] = neg
            return _
        lax.fori_loop(0, NPW, ib, None)

        def chunk(i, _):
            b = pl.multiple_of(a0 + i * C, C)
            pltpu.sync_copy(dst_h.at[pl.ds(b, C)], di_v)
            pltpu.sync_copy(m_h.at[pl.ds(b, C), :], m_v)

            def erow(e, _):
                g = plsc.load_gather(di_v, [jnp.full((16,), e, I32)])
                d = jnp.max(g, axis=0)
                ok = jnp.logical_and(d >= n0, d < n0 + NPW)

                @pl.when(ok)
                def _():
                    dl = d - n0
                    for j in range(H // 16):
                        sl = pl.ds(j * 16, 16)
                        acc_v[dl, sl] = jnp.maximum(acc_v[dl, sl], m_v[e, sl])
                return _
            lax.fori_loop(0, C, erow, None)
            return _
        lax.fori_loop(0, nch, chunk, None)
        pltpu.sync_copy(acc_v, out_h.at[pl.ds(n0, NPW)])

    return k(m, dstp, es)


def _pad_rows(a, rows, val=0):
    pad = jnp.full((rows - a.shape[0],) + a.shape[1:], val, a.dtype)
    return jnp.concatenate([a, pad], axis=0)


def _w2(l, fin_pad=None, fout_pad=None):
    w, b = l["w"].astype(F32), l["b"].astype(F32)
    if fin_pad is not None and w.shape[0] < fin_pad:
        w = jnp.concatenate(
            [w, jnp.zeros((fin_pad - w.shape[0], w.shape[1]), F32)], axis=0)
    if fout_pad is not None and w.shape[1] < fout_pad:
        w = jnp.concatenate(
            [w, jnp.zeros((w.shape[0], fout_pad - w.shape[1]), F32)], axis=1)
        b = jnp.concatenate([b, jnp.zeros((fout_pad - b.shape[0],), F32)])
    return w, b[None, :]


def _g2(bn):
    return jnp.stack([bn["g"].astype(F32), bn["b"].astype(F32)], axis=0)


def _ec_split(l, F, fin_pad=None):
    w = l["w"].astype(F32)
    wtop, wbot = w[:F], w[F:]
    if fin_pad is not None and F < fin_pad:
        z = jnp.zeros((fin_pad - F, w.shape[1]), F32)
        wtop = jnp.concatenate([wtop, z], axis=0)
        wbot = jnp.concatenate([wbot, z], axis=0)
    return wtop, wbot, l["b"].astype(F32)[None, :]


def kernel(params, pos, x_int, edge_index, edge_attr_int, batch, edge_index_e):
    p = params

    xi_p = _pad_rows(x_int.astype(I32), NP)
    pos_p = _pad_rows(
        jnp.concatenate([pos.astype(F32), jnp.zeros((NN, 13), F32)], axis=1),
        NP)
    ai_p = _pad_rows(
        jnp.concatenate([edge_attr_int.astype(I32),
                         jnp.zeros((EE, 1), I32)], axis=1), EP)
    src_p = _pad_rows(edge_index[0].astype(I32), EP, 0)
    dst_p = _pad_rows(edge_index[1].astype(I32), EP, NN)
    batch_p = _pad_rows(batch.astype(I32)[:, None], NP, GG)

    de = edge_index_e[1].astype(I32)
    se = edge_index_e[0].astype(I32)
    order = jnp.argsort(de)
    de_s = _pad_rows(de[order], EP, NN)
    se_s = _pad_rows(se[order], EP, 0)
    bounds = jnp.arange(0, NP + NPW, NPW, dtype=I32)
    es = jnp.searchsorted(de_s, bounds, side="left").astype(I32)
    es = jnp.concatenate([es, jnp.full((48 - 33,), EP, I32)])

    emb1 = p["emb1"].astype(F32)
    emb2 = _pad_rows(p["emb2"].astype(F32), 8)
    emb3 = _pad_rows(p["emb3"].astype(F32), 8)
    emb4 = _pad_rows(p["emb4"].astype(F32), 8)
    embp = jnp.concatenate(
        [_pad_rows(p["edge_emb"].astype(F32), 8), jnp.zeros((8, 1), F32)],
        axis=1)
    we1, be1 = _w2(p["lin_e1"], fout_pad=80)
    we2, be2 = _w2(p["lin_e2"])
    we3, be3 = _w2(p["lin_e3"])
    we4, be4 = _w2(p["lin_e4"])
    wa1, ba1 = _w2(p["nn1a"], fin_pad=80, fout_pad=80)
    wb1, bb1 = _w2(p["nn1b"], fin_pad=80)
    wa2, ba2 = _w2(p["nn2a"])
    wb2_, bb2 = _w2(p["nn2b"])
    wa3, ba3 = _w2(p["nn3a"])
    wb3, bb3 = _w2(p["nn3b"])
    wa4, ba4 = _w2(p["nn4a"])
    wb4, bb4 = _w2(p["nn4b"])
    eg1t, eg1b, eg1bias = _ec_split(p["eg1a"], 3, fin_pad=16)
    eg2t, eg2b, eg2bias = _ec_split(p["eg2a"], 64)
    eg3t, eg3b, eg3bias = _ec_split(p["eg3a"], 256)
    eg4t, eg4b, eg4bias = _ec_split(p["eg4a"], 256)
    weg1b, beg1b = _w2(p["eg1b"])
    weg2b, beg2b = _w2(p["eg2b"])
    weg3b, beg3b = _w2(p["eg3b"])
    weg4b, beg4b = _w2(p["eg4b"])

    x0, A1 = _embed_nodes(xi_p, pos_p, emb1, emb2, emb3, emb4, eg1t)
    eam1, eam2, eam3, eam4 = _embed_edges(
        ai_p, embp, we1, be1, we2, be2, we3, be3, we4, be4)

    def gine(xl, eaml, F, Fh, Fo, wa, ba, wb, bb, eps):
        parts = _gine_agg(xl, eaml, src_p, dst_p, F)
        y = _gine_mlp(xl, parts, wa, ba, wb, bb,
                      eps.astype(F32).reshape(1, 1), F, Fh, Fo)
        return y, _stats2(y, Fo)

    y1, st1 = gine(x0, eam1, 80, 80, 32, wa1, ba1, wb1, bb1, p["eps1"])
    x1 = _bn_apply(y1, st1, _g2(p["bng1"]), 32)
    y2, st2 = gine(x1, eam2, 32, 128, 32, wa2, ba2, wb2_, bb2, p["eps2"])
    x2 = _bn_apply(y2, st2, _g2(p["bng2"]), 32)
    y3, st3 = gine(x2, eam3, 32, 128, 32, wa3, ba3, wb3, bb3, p["eps3"])
    x3 = _bn_apply(y3, st3, _g2(p["bng3"]), 32)
    y4, st4 = gine(x3, eam4, 32, 32, 48, wa4, ba4, wb4, bb4, p["eps4"])
    x4 = _bn_apply(y4, st4, _g2(p["bng4"]), 48)

    def edgeconv(xn, Al, F, H, wbot, bias1, wegb, begb, bn_g2, proj):
        D, Ag = _edge_fetch(xn, Al, de_s, se_s, F, H)
        m = _edge_mlp(D, Ag, wbot, bias1, wegb, begb, F, H)
        o = _seg_max(m, de_s, es, H)
        st = _stats2(o, H, relu_in=True)
        return _bn_apply(o, st, bn_g2, H, relu_in=True, proj=proj)

    xe1, A2 = edgeconv(pos_p, A1, 16, 64, eg1b, eg1bias, weg1b, beg1b,
                       _g2(p["bn1"]), eg2t)
    xe2, A3 = edgeconv(xe1, A2, 64, 256, eg2b, eg2bias, weg2b, beg2b,
                       _g2(p["bn2"]), eg3t)
    xe3, A4 = edgeconv(xe2, A3, 256, 256, eg3b, eg3bias, weg3b, beg3b,
                       _g2(p["bn3"]), eg4t)
    xe4 = edgeconv(xe3, A4, 256, 208, eg4b, eg4bias, weg4b, beg4b,
                   _g2(p["bn4"]), None)

    return _pool_final(xe4, x4, batch_p, _g2(p["bn6"]))


# speedup vs baseline: 1.0139x; 1.0139x over previous
import functools

import jax
import jax.numpy as jnp
from jax import lax
from jax.experimental import pallas as pl
from jax.experimental.pallas import tpu as pltpu
from jax.experimental.pallas import tpu_sc as plsc

NN = 10000
EE = 160000
GG = 64
NP = 10240
EP = 163840
NW = 32
EPW = EP // NW
NPW = NP // NW
RB = 512
NB = NP // RB
F32 = jnp.float32
I32 = jnp.int32

_SC_PARAMS = pltpu.CompilerParams(use_tc_tiling_on_sc=False,
                                  needs_layout_passes=False)


def _fs(shape):
    return pl.BlockSpec(shape, lambda i: tuple(0 for _ in shape))


def _rows_mask(i):
    rows = lax.broadcasted_iota(I32, (RB, 1), 0) + i * RB
    return rows < NN


def _dot(a, b, hi=False):
    return jnp.dot(a, b, preferred_element_type=F32,
                   precision=(lax.Precision.HIGHEST if hi
                              else lax.Precision.DEFAULT))


def _embed_nodes(xi_p, pos_p, e1, e2, e3, e4, w1top):
    def body(xi_ref, pos_ref, e1_ref, e2_ref, e3_ref, e4_ref, wt_ref,
             x_ref, a_ref):
        xi = xi_ref[...]
        oh1 = (xi[:, 0:1] == lax.broadcasted_iota(I32, (RB, 16), 1)).astype(F32)
        oh2 = (xi[:, 1:2] == lax.broadcasted_iota(I32, (RB, 8), 1)).astype(F32)
        oh3 = (xi[:, 2:3] == lax.broadcasted_iota(I32, (RB, 8), 1)).astype(F32)
        oh4 = (xi[:, 3:4] == lax.broadcasted_iota(I32, (RB, 8), 1)).astype(F32)
        c1 = _dot(oh1, e1_ref[...], hi=True)
        c2 = (_dot(oh2, e2_ref[...], hi=True)
              + _dot(oh3, e3_ref[...], hi=True)
              + _dot(oh4, e4_ref[...], hi=True))
        c3 = (xi[:, 4:5] - 1).astype(F32)
        x_ref[...] = jnp.concatenate(
            [c1, c2, c3, jnp.zeros((RB, 15), F32)], axis=1)
        a_ref[...] = _dot(pos_ref[...], wt_ref[...])

    return pl.pallas_call(
        body, grid=(NB,),
        in_specs=[pl.BlockSpec((RB, 8), lambda i: (i, 0)),
                  pl.BlockSpec((RB, 16), lambda i: (i, 0)),
                  _fs((16, 32)), _fs((8, 32)), _fs((8, 32)), _fs((8, 32)),
                  _fs((16, 64))],
        out_specs=[pl.BlockSpec((RB, 80), lambda i: (i, 0)),
                   pl.BlockSpec((RB, 64), lambda i: (i, 0))],
        out_shape=[jax.ShapeDtypeStruct((NP, 80), F32),
                   jax.ShapeDtypeStruct((NP, 64), F32)],
    )(xi_p, pos_p, e1, e2, e3, e4, w1top)


def _embed_edges(ai_p, embp, w1, bb1, w2, bb2, w3, bb3, w4, bb4):
    def body(ai_ref, emb_ref, w1_ref, b1_ref, w2_ref, b2_ref, w3_ref, b3_ref,
             w4_ref, b4_ref, o1_ref, o2_ref, o3_ref, o4_ref):
        ai = ai_ref[...]
        oh = (ai[:, 0:1] == lax.broadcasted_iota(I32, (RB, 8), 1)).astype(F32)
        et = _dot(oh, emb_ref[...], hi=True)
        ea = jnp.concatenate([et[:, :15], (ai[:, 1:2] - 1).astype(F32)], axis=1)
        o1_ref[...] = _dot(ea, w1_ref[...]) + b1_ref[...]
        o2_ref[...] = _dot(ea, w2_ref[...]) + b2_ref[...]
        o3_ref[...] = _dot(ea, w3_ref[...]) + b3_ref[...]
        o4_ref[...] = _dot(ea, w4_ref[...]) + b4_ref[...]

    return pl.pallas_call(
        body, grid=(EP // RB,),
        in_specs=[pl.BlockSpec((RB, 8), lambda i: (i, 0)),
                  _fs((8, 16)), _fs((16, 80)), _fs((1, 80)),
                  _fs((16, 32)), _fs((1, 32)), _fs((16, 32)), _fs((1, 32)),
                  _fs((16, 32)), _fs((1, 32))],
        out_specs=[pl.BlockSpec((RB, 80), lambda i: (i, 0)),
                   pl.BlockSpec((RB, 32), lambda i: (i, 0)),
                   pl.BlockSpec((RB, 32), lambda i: (i, 0)),
                   pl.BlockSpec((RB, 32), lambda i: (i, 0))],
        out_shape=[jax.ShapeDtypeStruct((EP, 80), F32),
                   jax.ShapeDtypeStruct((EP, 32), F32),
                   jax.ShapeDtypeStruct((EP, 32), F32),
                   jax.ShapeDtypeStruct((EP, 32), F32)],
    )(ai_p, embp, w1, bb1, w2, bb2, w3, bb3, w4, bb4)


def _gine_mlp(x, parts, wa, ba, wb, bb, eps, F, Fh, Fo):
    def body(x_ref, p_ref, wa_ref, ba_ref, wb_ref, bb_ref, eps_ref, y_ref):
        h = (1.0 + eps_ref[0, 0]) * x_ref[...] + p_ref[0] + p_ref[1]
        t = jnp.maximum(_dot(h, wa_ref[...]) + ba_ref[...], 0.0)
        y_ref[...] = jnp.maximum(_dot(t, wb_ref[...]) + bb_ref[...], 0.0)

    return pl.pallas_call(
        body, grid=(NB,),
        in_specs=[pl.BlockSpec((RB, F), lambda i: (i, 0)),
                  pl.BlockSpec((2, RB, F), lambda i: (0, i, 0)),
                  _fs((F, Fh)), _fs((1, Fh)), _fs((Fh, Fo)), _fs((1, Fo)),
                  _fs((1, 1))],
        out_specs=pl.BlockSpec((RB, Fo), lambda i: (i, 0)),
        out_shape=jax.ShapeDtypeStruct((NP, Fo), F32),
    )(x, parts, wa, ba, wb, bb, eps)


def _stats2(y, F, relu_in=False):
    def body(y_ref, st_ref):
        i = pl.program_id(0)
        yv = y_ref[...]
        if relu_in:
            yv = jnp.maximum(yv, 0.0)
        mask = _rows_mask(i % NB)

        @pl.when(i == 0)
        def _():
            st_ref[...] = jnp.zeros((8, F), F32)

        @pl.when(i < NB)
        def _():
            ym = jnp.where(mask, yv, 0.0)
            st_ref[0:1, :] += jnp.sum(ym, axis=0, keepdims=True)

        @pl.when(i >= NB)
        def _():
            mean = st_ref[0:1, :] / NN
            d = jnp.where(mask, yv - mean, 0.0)
            st_ref[1:2, :] += jnp.sum(d * d, axis=0, keepdims=True)

    return pl.pallas_call(
        body, grid=(2 * NB,),
        in_specs=[pl.BlockSpec((RB, F), lambda i: (i % NB, 0))],
        out_specs=pl.BlockSpec((8, F), lambda i: (0, 0)),
        out_shape=jax.ShapeDtypeStruct((8, F), F32),
    )(y)


def _bn_apply(y, st, g2, F, relu_in=False, proj=None):
    H = proj.shape[1] if proj is not None else 0

    def body(*refs):
        if proj is not None:
            y_ref, st_ref, g_ref, wt_ref, xn_ref, a_ref = refs
        else:
            y_ref, st_ref, g_ref, xn_ref = refs
        mean = st_ref[0:1, :] / NN
        var = st_ref[1:2, :] / NN
        rs = lax.rsqrt(var + 1e-5)
        yv = y_ref[...]
        if relu_in:
            yv = jnp.maximum(yv, 0.0)
        xn = (yv - mean) * rs * g_ref[0:1, :] + g_ref[1:2, :]
        xn_ref[...] = xn
        if proj is not None:
            a_ref[...] = _dot(xn, wt_ref[...])

    in_specs = [pl.BlockSpec((RB, F), lambda i: (i, 0)),
                _fs((8, F)), _fs((2, F))]
    out_specs = [pl.BlockSpec((RB, F), lambda i: (i, 0))]
    out_shape = [jax.ShapeDtypeStruct((NP, F), F32)]
    args = [y, st, g2]
    if proj is not None:
        in_specs += [_fs((F, H))]
        out_specs += [pl.BlockSpec((RB, H), lambda i: (i, 0))]
        out_shape += [jax.ShapeDtypeStruct((NP, H), F32)]
        args += [proj]
    res = pl.pallas_call(body, grid=(NB,), in_specs=in_specs,
                         out_specs=out_specs, out_shape=out_shape)(*args)
    return res if proj is not None else res[0]


def _edge_mlp(D, Ag, wbot, b1, wb, b2, Fd, H):
    def body(d_ref, ag_ref, wbot_ref, b1_ref, wb_ref, b2_ref, m_ref):
        rh = jnp.maximum(
            ag_ref[...] + _dot(d_ref[...], wbot_ref[...]) + b1_ref[...], 0.0)
        m_ref[...] = _dot(rh, wb_ref[...]) + b2_ref[...]

    return pl.pallas_call(
        body, grid=(EP // RB,),
        in_specs=[pl.BlockSpec((RB, Fd), lambda i: (i, 0)),
                  pl.BlockSpec((RB, H), lambda i: (i, 0)),
                  _fs((Fd, H)), _fs((1, H)), _fs((H, H)), _fs((1, H))],
        out_specs=pl.BlockSpec((RB, H), lambda i: (i, 0)),
        out_shape=jax.ShapeDtypeStruct((EP, H), F32),
    )(D, Ag, wbot, b1, wb, b2)


def _pool_final(xe4, x4, batch_p, g2):
    def body(xe_ref, x4_ref, b_ref, g_ref, o_ref, acc):
        i = pl.program_id(0)

        @pl.when(i == 0)
        def _():
            acc[...] = jnp.zeros((GG, 256), F32)

        oh = (b_ref[...] == lax.broadcasted_iota(I32, (RB, GG), 1)).astype(F32)
        xcat = jnp.concatenate([xe_ref[...], x4_ref[...]], axis=1)
        acc[...] += lax.dot_general(oh, xcat, (((0,), (0,)), ((), ())),
                                    preferred_element_type=F32,
                                    precision=lax.Precision.HIGHEST)

        @pl.when(i == NB - 1)
        def _():
            pv = jnp.maximum(acc[...], 0.0)
            m = jnp.sum(pv, axis=0, keepdims=True) / GG
            d = pv - m
            v = jnp.sum(d * d, axis=0, keepdims=True) / GG
            rs = lax.rsqrt(v + 1e-5)
            o_ref[...] = d * rs * g_ref[0:1, :] + g_ref[1:2, :]

    return pl.pallas_call(
        body, grid=(NB,),
        in_specs=[pl.BlockSpec((RB, 208), lambda i: (i, 0)),
                  pl.BlockSpec((RB, 48), lambda i: (i, 0)),
                  pl.BlockSpec((RB, 1), lambda i: (i, 0)),
                  _fs((2, 256))],
        out_specs=pl.BlockSpec((GG, 256), lambda i: (0, 0)),
        out_shape=jax.ShapeDtypeStruct((GG, 256), F32),
        scratch_shapes=[pltpu.VMEM((GG, 256), F32)],
    )(xe4, x4, batch_p, g2)


@functools.cache
def _mesh():
    return plsc.VectorSubcoreMesh(core_axis_name="c", subcore_axis_name="s",
                                  num_cores=2, num_subcores=16)


def _gine_agg(x, eam, srcp, dstp, F):
    C = 128
    NSUB = 16
    ZR = NP // NSUB

    @functools.partial(
        pl.kernel,
        out_type=jax.ShapeDtypeStruct((2, NP, F), F32),
        mesh=_mesh(),
        compiler_params=_SC_PARAMS,
        scratch_types=[
            pltpu.VMEM((C,), I32),
            pltpu.VMEM((C,), I32),
            pltpu.VMEM((C, F), F32),
            pltpu.VMEM((C, F), F32),
            pltpu.VMEM((C, F), F32),
            pltpu.VMEM_SHARED((NP, F), F32),
            pltpu.SemaphoreType.DMA,
        ])
    def k(x_h, eam_h, src_h, dst_h, out_h, src_v, dst_v, rows_v, eam_v,
          msg_v, acc, sem):
        c = lax.axis_index("c")
        s = lax.axis_index("s")
        wid = s * 2 + c

        @plsc.parallel_loop(0, C, unroll=8)
        def zr_row(i):
            for j in range(F // 16):
                msg_v[i, pl.ds(j * 16, 16)] = jnp.zeros((16,), F32)

        def zcopy(i, _):
            pltpu.sync_copy(msg_v, acc.at[pl.ds(s * ZR + i * C, C)])
            return _
        lax.fori_loop(0, ZR // C, zcopy, None)
        plsc.subcore_barrier()

        ebase = wid * EPW

        def chunk(i, _):
            b = ebase + i * C
            pltpu.sync_copy(src_h.at[pl.ds(b, C)], src_v)
            pltpu.sync_copy(dst_h.at[pl.ds(b, C)], dst_v)
            pltpu.async_copy(x_h.at[src_v], rows_v, sem).wait()
            pltpu.sync_copy(eam_h.at[pl.ds(b, C), :], eam_v)

            @plsc.parallel_loop(0, C, unroll=8)
            def erow(e):
                for j in range(F // 16):
                    sl = pl.ds(j * 16, 16)
                    msg_v[e, sl] = jnp.maximum(rows_v[e, sl] + eam_v[e, sl],
                                               0.0)
            pltpu.sync_copy(msg_v, acc.at[dst_v], add=True)
            return _
        lax.fori_loop(0, EPW // C, chunk, None)
        plsc.subcore_barrier()

        def wb(i, _):
            r = s * ZR + i * C
            pltpu.sync_copy(acc.at[pl.ds(r, C)], msg_v)
            pltpu.sync_copy(msg_v, out_h.at[c, pl.ds(r, C)])
            return _
        lax.fori_loop(0, ZR // C, wb, None)

    return k(x, eam, srcp, dstp)


def _edge_fetch(x, A, de_s, se_s, F, H):
    C = 64

    @functools.partial(
        pl.kernel,
        out_type=(jax.ShapeDtypeStruct((EP, F), F32),
                  jax.ShapeDtypeStruct((EP, H), F32)),
        mesh=_mesh(),
        compiler_params=_SC_PARAMS,
        scratch_types=[
            pltpu.VMEM((C,), I32),
            pltpu.VMEM((C,), I32),
            pltpu.VMEM((C, F), F32),
            pltpu.VMEM((C, F), F32),
            pltpu.VMEM((C, H), F32),
            pltpu.SemaphoreType.DMA,
            pltpu.SemaphoreType.DMA,
            pltpu.SemaphoreType.DMA,
        ])
    def k(x_h, a_h, de_h, se_h, d_out, ag_out, di_v, si_v, xs_v, xd_v, ag_v,
          sem1, sem2, sem3):
        c = lax.axis_index("c")
        s = lax.axis_index("s")
        wid = s * 2 + c
        ebase = wid * EPW

        def chunk(i, _):
            b = ebase + i * C
            pltpu.sync_copy(de_h.at[pl.ds(b, C)], di_v)
            pltpu.sync_copy(se_h.at[pl.ds(b, C)], si_v)
            c1 = pltpu.async_copy(x_h.at[si_v], xs_v, sem1)
            c2 = pltpu.async_copy(x_h.at[di_v], xd_v, sem2)
            c3 = pltpu.async_copy(a_h.at[di_v], ag_v, sem3)
            c1.wait()
            c2.wait()
            c3.wait()

            @plsc.parallel_loop(0, C, unroll=8)
            def erow(e):
                for j in range(F // 16):
                    sl = pl.ds(j * 16, 16)
                    xs_v[e, sl] = xs_v[e, sl] - xd_v[e, sl]
            pltpu.sync_copy(xs_v, d_out.at[pl.ds(b, C), :])
            pltpu.sync_copy(ag_v, ag_out.at[pl.ds(b, C), :])
            return _
        lax.fori_loop(0, EPW // C, chunk, None)

    return k(x, A, de_s, se_s)


def _seg_max(m, dstp, es, H):
    C = 64

    @functools.partial(
        pl.kernel,
        out_type=jax.ShapeDtypeStruct((NP, H), F32),
        mesh=_mesh(),
        compiler_params=_SC_PARAMS,
        scratch_types=[
            pltpu.VMEM((C + 16,), I32),
            pltpu.VMEM((C, H), F32),
            pltpu.VMEM((NPW, H), F32),
            pltpu.VMEM((48,), I32),
        ])
    def k(m_h, dst_h, es_h, out_h, di_v, m_v, acc_v, es_v):
        c = lax.axis_index("c")
        s = lax.axis_index("s")
        wid = s * 2 + c
        n0 = wid * NPW
        pltpu.sync_copy(es_h, es_v)

        e0 = es_v[pl.ds(wid, 16)][0]
        e1 = es_v[pl.ds(wid + 1, 16)][0]
        a0 = jnp.bitwise_and(e0, jnp.int32(-C))
        nch = (e1 - a0 + (C - 1)) // C

        neg = jnp.full((16,), -jnp.inf, F32)

        @plsc.parallel_loop(0, NPW, unroll=8)
        def ib(i):
            for j in range(H // 16):
                acc_v[i, pl.ds(j * 16, 16)] = neg

        def chunk(i, _):
            b = pl.multiple_of(a0 + i * C, C)
            pltpu.sync_copy(dst_h.at[pl.ds(b, C)], di_v.at[pl.ds(0, C)])
            pltpu.sync_copy(m_h.at[pl.ds(b, C), :], m_v)

            def erow(e, _):
                d = di_v[pl.ds(e, 16)][0]
                ok = jnp.logical_and(d >= n0, d < n0 + NPW)

                @pl.when(ok)
                def _():
                    dl = d - n0
                    for j in range(H // 16):
                        sl = pl.ds(j * 16, 16)
                        acc_v[dl, sl] = jnp.maximum(acc_v[dl, sl], m_v[e, sl])
                return _
            lax.fori_loop(0, C, erow, None)
            return _
        lax.fori_loop(0, nch, chunk, None)
        pltpu.sync_copy(acc_v, out_h.at[pl.ds(n0, NPW)])

    return k(m, dstp, es)


def _pad_rows(a, rows, val=0):
    pad = jnp.full((rows - a.shape[0],) + a.shape[1:], val, a.dtype)
    return jnp.concatenate([a, pad], axis=0)


def _w2(l, fin_pad=None, fout_pad=None):
    w, b = l["w"].astype(F32), l["b"].astype(F32)
    if fin_pad is not None and w.shape[0] < fin_pad:
        w = jnp.concatenate(
            [w, jnp.zeros((fin_pad - w.shape[0], w.shape[1]), F32)], axis=0)
    if fout_pad is not None and w.shape[1] < fout_pad:
        w = jnp.concatenate(
            [w, jnp.zeros((w.shape[0], fout_pad - w.shape[1]), F32)], axis=1)
        b = jnp.concatenate([b, jnp.zeros((fout_pad - b.shape[0],), F32)])
    return w, b[None, :]


def _g2(bn):
    return jnp.stack([bn["g"].astype(F32), bn["b"].astype(F32)], axis=0)


def _ec_split(l, F, fin_pad=None):
    w = l["w"].astype(F32)
    wtop, wbot = w[:F], w[F:]
    if fin_pad is not None and F < fin_pad:
        z = jnp.zeros((fin_pad - F, w.shape[1]), F32)
        wtop = jnp.concatenate([wtop, z], axis=0)
        wbot = jnp.concatenate([wbot, z], axis=0)
    return wtop, wbot, l["b"].astype(F32)[None, :]


def kernel(params, pos, x_int, edge_index, edge_attr_int, batch, edge_index_e):
    p = params

    xi_p = _pad_rows(x_int.astype(I32), NP)
    pos_p = _pad_rows(
        jnp.concatenate([pos.astype(F32), jnp.zeros((NN, 13), F32)], axis=1),
        NP)
    ai_p = _pad_rows(
        jnp.concatenate([edge_attr_int.astype(I32),
                         jnp.zeros((EE, 1), I32)], axis=1), EP)
    src_p = _pad_rows(edge_index[0].astype(I32), EP, 0)
    dst_p = _pad_rows(edge_index[1].astype(I32), EP, NN)
    batch_p = _pad_rows(batch.astype(I32)[:, None], NP, GG)

    de = edge_index_e[1].astype(I32)
    se = edge_index_e[0].astype(I32)
    order = jnp.argsort(de)
    de_s = _pad_rows(de[order], EP, NN)
    se_s = _pad_rows(se[order], EP, 0)
    bounds = jnp.arange(0, NP + NPW, NPW, dtype=I32)
    es = jnp.searchsorted(de_s, bounds, side="left").astype(I32)
    es = jnp.concatenate([es, jnp.full((48 - 33,), EP, I32)])

    emb1 = p["emb1"].astype(F32)
    emb2 = _pad_rows(p["emb2"].astype(F32), 8)
    emb3 = _pad_rows(p["emb3"].astype(F32), 8)
    emb4 = _pad_rows(p["emb4"].astype(F32), 8)
    embp = jnp.concatenate(
        [_pad_rows(p["edge_emb"].astype(F32), 8), jnp.zeros((8, 1), F32)],
        axis=1)
    we1, be1 = _w2(p["lin_e1"], fout_pad=80)
    we2, be2 = _w2(p["lin_e2"])
    we3, be3 = _w2(p["lin_e3"])
    we4, be4 = _w2(p["lin_e4"])
    wa1, ba1 = _w2(p["nn1a"], fin_pad=80, fout_pad=80)
    wb1, bb1 = _w2(p["nn1b"], fin_pad=80)
    wa2, ba2 = _w2(p["nn2a"])
    wb2_, bb2 = _w2(p["nn2b"])
    wa3, ba3 = _w2(p["nn3a"])
    wb3, bb3 = _w2(p["nn3b"])
    wa4, ba4 = _w2(p["nn4a"])
    wb4, bb4 = _w2(p["nn4b"])
    eg1t, eg1b, eg1bias = _ec_split(p["eg1a"], 3, fin_pad=16)
    eg2t, eg2b, eg2bias = _ec_split(p["eg2a"], 64)
    eg3t, eg3b, eg3bias = _ec_split(p["eg3a"], 256)
    eg4t, eg4b, eg4bias = _ec_split(p["eg4a"], 256)
    weg1b, beg1b = _w2(p["eg1b"])
    weg2b, beg2b = _w2(p["eg2b"])
    weg3b, beg3b = _w2(p["eg3b"])
    weg4b, beg4b = _w2(p["eg4b"])

    x0, A1 = _embed_nodes(xi_p, pos_p, emb1, emb2, emb3, emb4, eg1t)
    eam1, eam2, eam3, eam4 = _embed_edges(
        ai_p, embp, we1, be1, we2, be2, we3, be3, we4, be4)

    def gine(xl, eaml, F, Fh, Fo, wa, ba, wb, bb, eps):
        parts = _gine_agg(xl, eaml, src_p, dst_p, F)
        y = _gine_mlp(xl, parts, wa, ba, wb, bb,
                      eps.astype(F32).reshape(1, 1), F, Fh, Fo)
        return y, _stats2(y, Fo)

    y1, st1 = gine(x0, eam1, 80, 80, 32, wa1, ba1, wb1, bb1, p["eps1"])
    x1 = _bn_apply(y1, st1, _g2(p["bng1"]), 32)
    y2, st2 = gine(x1, eam2, 32, 128, 32, wa2, ba2, wb2_, bb2, p["eps2"])
    x2 = _bn_apply(y2, st2, _g2(p["bng2"]), 32)
    y3, st3 = gine(x2, eam3, 32, 128, 32, wa3, ba3, wb3, bb3, p["eps3"])
    x3 = _bn_apply(y3, st3, _g2(p["bng3"]), 32)
    y4, st4 = gine(x3, eam4, 32, 32, 48, wa4, ba4, wb4, bb4, p["eps4"])
    x4 = _bn_apply(y4, st4, _g2(p["bng4"]), 48)

    def edgeconv(xn, Al, F, H, wbot, bias1, wegb, begb, bn_g2, proj):
        D, Ag = _edge_fetch(xn, Al, de_s, se_s, F, H)
        m = _edge_mlp(D, Ag, wbot, bias1, wegb, begb, F, H)
        o = _seg_max(m, de_s, es, H)
        st = _stats2(o, H, relu_in=True)
        return _bn_apply(o, st, bn_g2, H, relu_in=True, proj=proj)

    xe1, A2 = edgeconv(pos_p, A1, 16, 64, eg1b, eg1bias, weg1b, beg1b,
                       _g2(p["bn1"]), eg2t)
    xe2, A3 = edgeconv(xe1, A2, 64, 256, eg2b, eg2bias, weg2b, beg2b,
                       _g2(p["bn2"]), eg3t)
    xe3, A4 = edgeconv(xe2, A3, 256, 256, eg3b, eg3bias, weg3b, beg3b,
                       _g2(p["bn3"]), eg4t)
    xe4 = edgeconv(xe3, A4, 256, 208, eg4b, eg4bias, weg4b, beg4b,
                   _g2(p["bn4"]), None)

    return _pool_final(xe4, x4, batch_p, _g2(p["bn6"]))


# speedup vs baseline: 1.1015x; 1.0864x over previous
import functools

import jax
import jax.numpy as jnp
from jax import lax
from jax.experimental import pallas as pl
from jax.experimental.pallas import tpu as pltpu
from jax.experimental.pallas import tpu_sc as plsc

NN = 10000
EE = 160000
GG = 64
NP = 10240
EP = 163840
NW = 32
EPW = EP // NW
NPW = NP // NW
RB = 512
NB = NP // RB
F32 = jnp.float32
I32 = jnp.int32

_SC_PARAMS = pltpu.CompilerParams(use_tc_tiling_on_sc=False,
                                  needs_layout_passes=False)


def _fs(shape):
    return pl.BlockSpec(shape, lambda i: tuple(0 for _ in shape))


def _rows_mask(i):
    rows = lax.broadcasted_iota(I32, (RB, 1), 0) + i * RB
    return rows < NN


def _dot(a, b, hi=False):
    return jnp.dot(a, b, preferred_element_type=F32,
                   precision=(lax.Precision.HIGHEST if hi
                              else lax.Precision.DEFAULT))


def _embed_nodes(xi_p, pos_p, e1, e2, e3, e4, w1top):
    def body(xi_ref, pos_ref, e1_ref, e2_ref, e3_ref, e4_ref, wt_ref,
             x_ref, a_ref):
        xi = xi_ref[...]
        oh1 = (xi[:, 0:1] == lax.broadcasted_iota(I32, (RB, 16), 1)).astype(F32)
        oh2 = (xi[:, 1:2] == lax.broadcasted_iota(I32, (RB, 8), 1)).astype(F32)
        oh3 = (xi[:, 2:3] == lax.broadcasted_iota(I32, (RB, 8), 1)).astype(F32)
        oh4 = (xi[:, 3:4] == lax.broadcasted_iota(I32, (RB, 8), 1)).astype(F32)
        c1 = _dot(oh1, e1_ref[...], hi=True)
        c2 = (_dot(oh2, e2_ref[...], hi=True)
              + _dot(oh3, e3_ref[...], hi=True)
              + _dot(oh4, e4_ref[...], hi=True))
        c3 = (xi[:, 4:5] - 1).astype(F32)
        x_ref[...] = jnp.concatenate(
            [c1, c2, c3, jnp.zeros((RB, 15), F32)], axis=1)
        a_ref[...] = _dot(pos_ref[...], wt_ref[...])

    return pl.pallas_call(
        body, grid=(NB,),
        in_specs=[pl.BlockSpec((RB, 8), lambda i: (i, 0)),
                  pl.BlockSpec((RB, 16), lambda i: (i, 0)),
                  _fs((16, 32)), _fs((8, 32)), _fs((8, 32)), _fs((8, 32)),
                  _fs((16, 64))],
        out_specs=[pl.BlockSpec((RB, 80), lambda i: (i, 0)),
                   pl.BlockSpec((RB, 64), lambda i: (i, 0))],
        out_shape=[jax.ShapeDtypeStruct((NP, 80), F32),
                   jax.ShapeDtypeStruct((NP, 64), F32)],
    )(xi_p, pos_p, e1, e2, e3, e4, w1top)


def _embed_edges(ai_p, embp, w1, bb1, w2, bb2, w3, bb3, w4, bb4):
    def body(ai_ref, emb_ref, w1_ref, b1_ref, w2_ref, b2_ref, w3_ref, b3_ref,
             w4_ref, b4_ref, o1_ref, o2_ref, o3_ref, o4_ref):
        ai = ai_ref[...]
        oh = (ai[:, 0:1] == lax.broadcasted_iota(I32, (RB, 8), 1)).astype(F32)
        et = _dot(oh, emb_ref[...], hi=True)
        ea = jnp.concatenate([et[:, :15], (ai[:, 1:2] - 1).astype(F32)], axis=1)
        o1_ref[...] = _dot(ea, w1_ref[...]) + b1_ref[...]
        o2_ref[...] = _dot(ea, w2_ref[...]) + b2_ref[...]
        o3_ref[...] = _dot(ea, w3_ref[...]) + b3_ref[...]
        o4_ref[...] = _dot(ea, w4_ref[...]) + b4_ref[...]

    return pl.pallas_call(
        body, grid=(EP // RB,),
        in_specs=[pl.BlockSpec((RB, 8), lambda i: (i, 0)),
                  _fs((8, 16)), _fs((16, 80)), _fs((1, 80)),
                  _fs((16, 32)), _fs((1, 32)), _fs((16, 32)), _fs((1, 32)),
                  _fs((16, 32)), _fs((1, 32))],
        out_specs=[pl.BlockSpec((RB, 80), lambda i: (i, 0)),
                   pl.BlockSpec((RB, 32), lambda i: (i, 0)),
                   pl.BlockSpec((RB, 32), lambda i: (i, 0)),
                   pl.BlockSpec((RB, 32), lambda i: (i, 0))],
        out_shape=[jax.ShapeDtypeStruct((EP, 80), F32),
                   jax.ShapeDtypeStruct((EP, 32), F32),
                   jax.ShapeDtypeStruct((EP, 32), F32),
                   jax.ShapeDtypeStruct((EP, 32), F32)],
    )(ai_p, embp, w1, bb1, w2, bb2, w3, bb3, w4, bb4)


def _gine_mlp(x, parts, wa, ba, wb, bb, eps, F, Fh, Fo):
    def body(x_ref, p_ref, wa_ref, ba_ref, wb_ref, bb_ref, eps_ref, y_ref):
        h = (1.0 + eps_ref[0, 0]) * x_ref[...] + p_ref[0] + p_ref[1]
        t = jnp.maximum(_dot(h, wa_ref[...]) + ba_ref[...], 0.0)
        y_ref[...] = jnp.maximum(_dot(t, wb_ref[...]) + bb_ref[...], 0.0)

    return pl.pallas_call(
        body, grid=(NB,),
        in_specs=[pl.BlockSpec((RB, F), lambda i: (i, 0)),
                  pl.BlockSpec((2, RB, F), lambda i: (0, i, 0)),
                  _fs((F, Fh)), _fs((1, Fh)), _fs((Fh, Fo)), _fs((1, Fo)),
                  _fs((1, 1))],
        out_specs=pl.BlockSpec((RB, Fo), lambda i: (i, 0)),
        out_shape=jax.ShapeDtypeStruct((NP, Fo), F32),
    )(x, parts, wa, ba, wb, bb, eps)


def _stats2(y, F, relu_in=False):
    def body(y_ref, st_ref):
        i = pl.program_id(0)
        yv = y_ref[...]
        if relu_in:
            yv = jnp.maximum(yv, 0.0)
        mask = _rows_mask(i % NB)

        @pl.when(i == 0)
        def _():
            st_ref[...] = jnp.zeros((8, F), F32)

        @pl.when(i < NB)
        def _():
            ym = jnp.where(mask, yv, 0.0)
            st_ref[0:1, :] += jnp.sum(ym, axis=0, keepdims=True)

        @pl.when(i >= NB)
        def _():
            mean = st_ref[0:1, :] / NN
            d = jnp.where(mask, yv - mean, 0.0)
            st_ref[1:2, :] += jnp.sum(d * d, axis=0, keepdims=True)

    return pl.pallas_call(
        body, grid=(2 * NB,),
        in_specs=[pl.BlockSpec((RB, F), lambda i: (i % NB, 0))],
        out_specs=pl.BlockSpec((8, F), lambda i: (0, 0)),
        out_shape=jax.ShapeDtypeStruct((8, F), F32),
    )(y)


def _bn_apply(y, st, g2, F, relu_in=False, proj=None):
    H = proj.shape[1] if proj is not None else 0

    def body(*refs):
        if proj is not None:
            y_ref, st_ref, g_ref, wt_ref, xn_ref, a_ref = refs
        else:
            y_ref, st_ref, g_ref, xn_ref = refs
        mean = st_ref[0:1, :] / NN
        var = st_ref[1:2, :] / NN
        rs = lax.rsqrt(var + 1e-5)
        yv = y_ref[...]
        if relu_in:
            yv = jnp.maximum(yv, 0.0)
        xn = (yv - mean) * rs * g_ref[0:1, :] + g_ref[1:2, :]
        xn_ref[...] = xn
        if proj is not None:
            a_ref[...] = _dot(xn, wt_ref[...])

    in_specs = [pl.BlockSpec((RB, F), lambda i: (i, 0)),
                _fs((8, F)), _fs((2, F))]
    out_specs = [pl.BlockSpec((RB, F), lambda i: (i, 0))]
    out_shape = [jax.ShapeDtypeStruct((NP, F), F32)]
    args = [y, st, g2]
    if proj is not None:
        in_specs += [_fs((F, H))]
        out_specs += [pl.BlockSpec((RB, H), lambda i: (i, 0))]
        out_shape += [jax.ShapeDtypeStruct((NP, H), F32)]
        args += [proj]
    res = pl.pallas_call(body, grid=(NB,), in_specs=in_specs,
                         out_specs=out_specs, out_shape=out_shape)(*args)
    return res if proj is not None else res[0]


def _edge_mlp(D, Ag, wbot, b1, wb, b2, Fd, H):
    def body(d_ref, ag_ref, wbot_ref, b1_ref, wb_ref, b2_ref, m_ref):
        rh = jnp.maximum(
            ag_ref[...] + _dot(d_ref[...], wbot_ref[...]) + b1_ref[...], 0.0)
        m_ref[...] = _dot(rh, wb_ref[...]) + b2_ref[...]

    return pl.pallas_call(
        body, grid=(EP // RB,),
        in_specs=[pl.BlockSpec((RB, Fd), lambda i: (i, 0)),
                  pl.BlockSpec((RB, H), lambda i: (i, 0)),
                  _fs((Fd, H)), _fs((1, H)), _fs((H, H)), _fs((1, H))],
        out_specs=pl.BlockSpec((RB, H), lambda i: (i, 0)),
        out_shape=jax.ShapeDtypeStruct((EP, H), F32),
    )(D, Ag, wbot, b1, wb, b2)


def _pool_final(xe4, x4, batch_p, g2):
    def body(xe_ref, x4_ref, b_ref, g_ref, o_ref, acc):
        i = pl.program_id(0)

        @pl.when(i == 0)
        def _():
            acc[...] = jnp.zeros((GG, 256), F32)

        oh = (b_ref[...] == lax.broadcasted_iota(I32, (RB, GG), 1)).astype(F32)
        xcat = jnp.concatenate([xe_ref[...], x4_ref[...]], axis=1)
        acc[...] += lax.dot_general(oh, xcat, (((0,), (0,)), ((), ())),
                                    preferred_element_type=F32,
                                    precision=lax.Precision.HIGHEST)

        @pl.when(i == NB - 1)
        def _():
            pv = jnp.maximum(acc[...], 0.0)
            m = jnp.sum(pv, axis=0, keepdims=True) / GG
            d = pv - m
            v = jnp.sum(d * d, axis=0, keepdims=True) / GG
            rs = lax.rsqrt(v + 1e-5)
            o_ref[...] = d * rs * g_ref[0:1, :] + g_ref[1:2, :]

    return pl.pallas_call(
        body, grid=(NB,),
        in_specs=[pl.BlockSpec((RB, 208), lambda i: (i, 0)),
                  pl.BlockSpec((RB, 48), lambda i: (i, 0)),
                  pl.BlockSpec((RB, 1), lambda i: (i, 0)),
                  _fs((2, 256))],
        out_specs=pl.BlockSpec((GG, 256), lambda i: (0, 0)),
        out_shape=jax.ShapeDtypeStruct((GG, 256), F32),
        scratch_shapes=[pltpu.VMEM((GG, 256), F32)],
    )(xe4, x4, batch_p, g2)


@functools.cache
def _mesh():
    return plsc.VectorSubcoreMesh(core_axis_name="c", subcore_axis_name="s",
                                  num_cores=2, num_subcores=16)


def _gine_agg(x, eam, srcp, dstp, F):
    C = 128
    NSUB = 16
    ZR = NP // NSUB

    @functools.partial(
        pl.kernel,
        out_type=jax.ShapeDtypeStruct((2, NP, F), F32),
        mesh=_mesh(),
        compiler_params=_SC_PARAMS,
        scratch_types=[
            pltpu.VMEM((C,), I32),
            pltpu.VMEM((C,), I32),
            pltpu.VMEM((C, F), F32),
            pltpu.VMEM((C, F), F32),
            pltpu.VMEM((C, F), F32),
            pltpu.VMEM_SHARED((NP, F), F32),
            pltpu.SemaphoreType.DMA,
        ])
    def k(x_h, eam_h, src_h, dst_h, out_h, src_v, dst_v, rows_v, eam_v,
          msg_v, acc, sem):
        c = lax.axis_index("c")
        s = lax.axis_index("s")
        wid = s * 2 + c

        @plsc.parallel_loop(0, C, unroll=8)
        def zr_row(i):
            for j in range(F // 16):
                msg_v[i, pl.ds(j * 16, 16)] = jnp.zeros((16,), F32)

        def zcopy(i, _):
            pltpu.sync_copy(msg_v, acc.at[pl.ds(s * ZR + i * C, C)])
            return _
        lax.fori_loop(0, ZR // C, zcopy, None)
        plsc.subcore_barrier()

        ebase = wid * EPW

        def chunk(i, _):
            b = ebase + i * C
            pltpu.sync_copy(src_h.at[pl.ds(b, C)], src_v)
            pltpu.sync_copy(dst_h.at[pl.ds(b, C)], dst_v)
            pltpu.async_copy(x_h.at[src_v], rows_v, sem).wait()
            pltpu.sync_copy(eam_h.at[pl.ds(b, C), :], eam_v)

            @plsc.parallel_loop(0, C, unroll=8)
            def erow(e):
                for j in range(F // 16):
                    sl = pl.ds(j * 16, 16)
                    msg_v[e, sl] = jnp.maximum(rows_v[e, sl] + eam_v[e, sl],
                                               0.0)
            pltpu.sync_copy(msg_v, acc.at[dst_v], add=True)
            return _
        lax.fori_loop(0, EPW // C, chunk, None)
        plsc.subcore_barrier()

        def wb(i, _):
            r = s * ZR + i * C
            pltpu.sync_copy(acc.at[pl.ds(r, C)], msg_v)
            pltpu.sync_copy(msg_v, out_h.at[c, pl.ds(r, C)])
            return _
        lax.fori_loop(0, ZR // C, wb, None)

    return k(x, eam, srcp, dstp)


def _edge_fetch(x, A, de_s, se_s, F, H):
    C = 64
    NCH = EPW // C

    @functools.partial(
        pl.kernel,
        out_type=(jax.ShapeDtypeStruct((EP, F), F32),
                  jax.ShapeDtypeStruct((EP, H), F32)),
        mesh=_mesh(),
        compiler_params=_SC_PARAMS,
        scratch_types=[
            pltpu.VMEM((2, C), I32),
            pltpu.VMEM((2, C), I32),
            pltpu.VMEM((2, C, F), F32),
            pltpu.VMEM((2, C, F), F32),
            pltpu.VMEM((2, C, H), F32),
            pltpu.SemaphoreType.DMA,
            pltpu.SemaphoreType.DMA,
            pltpu.SemaphoreType.DMA,
            pltpu.SemaphoreType.DMA,
            pltpu.SemaphoreType.DMA,
            pltpu.SemaphoreType.DMA,
        ])
    def k(x_h, a_h, de_h, se_h, d_out, ag_out, di_v, si_v, xs_v, xd_v, ag_v,
          s10, s20, s30, s11, s21, s31):
        c = lax.axis_index("c")
        s = lax.axis_index("s")
        wid = s * 2 + c
        ebase = wid * EPW
        sems = ((s10, s20, s30), (s11, s21, s31))

        def issue(ci, bi):
            b = ebase + ci * C
            pltpu.sync_copy(de_h.at[pl.ds(b, C)], di_v.at[bi])
            pltpu.sync_copy(se_h.at[pl.ds(b, C)], si_v.at[bi])
            pltpu.async_copy(x_h.at[si_v.at[bi]], xs_v.at[bi], sems[bi][0])
            pltpu.async_copy(x_h.at[di_v.at[bi]], xd_v.at[bi], sems[bi][1])
            pltpu.async_copy(a_h.at[di_v.at[bi]], ag_v.at[bi], sems[bi][2])

        def finish(ci, bi):
            pltpu.make_async_copy(
                x_h.at[si_v.at[bi]], xs_v.at[bi], sems[bi][0]).wait()
            pltpu.make_async_copy(
                x_h.at[di_v.at[bi]], xd_v.at[bi], sems[bi][1]).wait()
            pltpu.make_async_copy(
                a_h.at[di_v.at[bi]], ag_v.at[bi], sems[bi][2]).wait()
            b = ebase + ci * C

            @plsc.parallel_loop(0, C, unroll=8)
            def erow(e):
                for j in range(F // 16):
                    sl = pl.ds(j * 16, 16)
                    xs_v[bi, e, sl] = xs_v[bi, e, sl] - xd_v[bi, e, sl]
            pltpu.sync_copy(xs_v.at[bi], d_out.at[pl.ds(b, C), :])
            pltpu.sync_copy(ag_v.at[bi], ag_out.at[pl.ds(b, C), :])

        issue(0, 0)

        def pair(kk, _):
            i0 = kk * 2
            issue(i0 + 1, 1)
            finish(i0, 0)

            @pl.when(kk < NCH // 2 - 1)
            def _():
                issue(i0 + 2, 0)
            finish(i0 + 1, 1)
            return _
        lax.fori_loop(0, NCH // 2, pair, None)

    return k(x, A, de_s, se_s)


def _seg_max(m, dstp, es, H):
    C = 64

    @functools.partial(
        pl.kernel,
        out_type=jax.ShapeDtypeStruct((NP, H), F32),
        mesh=_mesh(),
        compiler_params=_SC_PARAMS,
        scratch_types=[
            pltpu.VMEM((C + 16,), I32),
            pltpu.VMEM((C, H), F32),
            pltpu.VMEM((NPW, H), F32),
            pltpu.VMEM((48,), I32),
        ])
    def k(m_h, dst_h, es_h, out_h, di_v, m_v, acc_v, es_v):
        c = lax.axis_index("c")
        s = lax.axis_index("s")
        wid = s * 2 + c
        n0 = wid * NPW
        pltpu.sync_copy(es_h, es_v)

        e0 = es_v[pl.ds(wid, 16)][0]
        e1 = es_v[pl.ds(wid + 1, 16)][0]
        a0 = jnp.bitwise_and(e0, jnp.int32(-C))
        nch = (e1 - a0 + (C - 1)) // C

        neg = jnp.full((16,), -jnp.inf, F32)

        @plsc.parallel_loop(0, NPW, unroll=8)
        def ib(i):
            for j in range(H // 16):
                acc_v[i, pl.ds(j * 16, 16)] = neg

        def chunk(i, _):
            b = pl.multiple_of(a0 + i * C, C)
            pltpu.sync_copy(dst_h.at[pl.ds(b, C)], di_v.at[pl.ds(0, C)])
            pltpu.sync_copy(m_h.at[pl.ds(b, C), :], m_v)

            def erow(e, _):
                d = di_v[pl.ds(e, 16)][0]
                ok = jnp.logical_and(d >= n0, d < n0 + NPW)

                @pl.when(ok)
                def _():
                    dl = d - n0
                    for j in range(H // 16):
                        sl = pl.ds(j * 16, 16)
                        acc_v[dl, sl] = jnp.maximum(acc_v[dl, sl], m_v[e, sl])
                return _
            lax.fori_loop(0, C, erow, None)
            return _
        lax.fori_loop(0, nch, chunk, None)
        pltpu.sync_copy(acc_v, out_h.at[pl.ds(n0, NPW)])

    return k(m, dstp, es)


def _pad_rows(a, rows, val=0):
    pad = jnp.full((rows - a.shape[0],) + a.shape[1:], val, a.dtype)
    return jnp.concatenate([a, pad], axis=0)


def _w2(l, fin_pad=None, fout_pad=None):
    w, b = l["w"].astype(F32), l["b"].astype(F32)
    if fin_pad is not None and w.shape[0] < fin_pad:
        w = jnp.concatenate(
            [w, jnp.zeros((fin_pad - w.shape[0], w.shape[1]), F32)], axis=0)
    if fout_pad is not None and w.shape[1] < fout_pad:
        w = jnp.concatenate(
            [w, jnp.zeros((w.shape[0], fout_pad - w.shape[1]), F32)], axis=1)
        b = jnp.concatenate([b, jnp.zeros((fout_pad - b.shape[0],), F32)])
    return w, b[None, :]


def _g2(bn):
    return jnp.stack([bn["g"].astype(F32), bn["b"].astype(F32)], axis=0)


def _ec_split(l, F, fin_pad=None):
    w = l["w"].astype(F32)
    wtop, wbot = w[:F], w[F:]
    if fin_pad is not None and F < fin_pad:
        z = jnp.zeros((fin_pad - F, w.shape[1]), F32)
        wtop = jnp.concatenate([wtop, z], axis=0)
        wbot = jnp.concatenate([wbot, z], axis=0)
    return wtop, wbot, l["b"].astype(F32)[None, :]


def kernel(params, pos, x_int, edge_index, edge_attr_int, batch, edge_index_e):
    p = params

    xi_p = _pad_rows(x_int.astype(I32), NP)
    pos_p = _pad_rows(
        jnp.concatenate([pos.astype(F32), jnp.zeros((NN, 13), F32)], axis=1),
        NP)
    ai_p = _pad_rows(
        jnp.concatenate([edge_attr_int.astype(I32),
                         jnp.zeros((EE, 1), I32)], axis=1), EP)
    src_p = _pad_rows(edge_index[0].astype(I32), EP, 0)
    dst_p = _pad_rows(edge_index[1].astype(I32), EP, NN)
    batch_p = _pad_rows(batch.astype(I32)[:, None], NP, GG)

    de = edge_index_e[1].astype(I32)
    se = edge_index_e[0].astype(I32)
    order = jnp.argsort(de)
    de_s = _pad_rows(de[order], EP, NN)
    se_s = _pad_rows(se[order], EP, 0)
    bounds = jnp.arange(0, NP + NPW, NPW, dtype=I32)
    es = jnp.searchsorted(de_s, bounds, side="left").astype(I32)
    es = jnp.concatenate([es, jnp.full((48 - 33,), EP, I32)])

    emb1 = p["emb1"].astype(F32)
    emb2 = _pad_rows(p["emb2"].astype(F32), 8)
    emb3 = _pad_rows(p["emb3"].astype(F32), 8)
    emb4 = _pad_rows(p["emb4"].astype(F32), 8)
    embp = jnp.concatenate(
        [_pad_rows(p["edge_emb"].astype(F32), 8), jnp.zeros((8, 1), F32)],
        axis=1)
    we1, be1 = _w2(p["lin_e1"], fout_pad=80)
    we2, be2 = _w2(p["lin_e2"])
    we3, be3 = _w2(p["lin_e3"])
    we4, be4 = _w2(p["lin_e4"])
    wa1, ba1 = _w2(p["nn1a"], fin_pad=80, fout_pad=80)
    wb1, bb1 = _w2(p["nn1b"], fin_pad=80)
    wa2, ba2 = _w2(p["nn2a"])
    wb2_, bb2 = _w2(p["nn2b"])
    wa3, ba3 = _w2(p["nn3a"])
    wb3, bb3 = _w2(p["nn3b"])
    wa4, ba4 = _w2(p["nn4a"])
    wb4, bb4 = _w2(p["nn4b"])
    eg1t, eg1b, eg1bias = _ec_split(p["eg1a"], 3, fin_pad=16)
    eg2t, eg2b, eg2bias = _ec_split(p["eg2a"], 64)
    eg3t, eg3b, eg3bias = _ec_split(p["eg3a"], 256)
    eg4t, eg4b, eg4bias = _ec_split(p["eg4a"], 256)
    weg1b, beg1b = _w2(p["eg1b"])
    weg2b, beg2b = _w2(p["eg2b"])
    weg3b, beg3b = _w2(p["eg3b"])
    weg4b, beg4b = _w2(p["eg4b"])

    x0, A1 = _embed_nodes(xi_p, pos_p, emb1, emb2, emb3, emb4, eg1t)
    eam1, eam2, eam3, eam4 = _embed_edges(
        ai_p, embp, we1, be1, we2, be2, we3, be3, we4, be4)

    def gine(xl, eaml, F, Fh, Fo, wa, ba, wb, bb, eps):
        parts = _gine_agg(xl, eaml, src_p, dst_p, F)
        y = _gine_mlp(xl, parts, wa, ba, wb, bb,
                      eps.astype(F32).reshape(1, 1), F, Fh, Fo)
        return y, _stats2(y, Fo)

    y1, st1 = gine(x0, eam1, 80, 80, 32, wa1, ba1, wb1, bb1, p["eps1"])
    x1 = _bn_apply(y1, st1, _g2(p["bng1"]), 32)
    y2, st2 = gine(x1, eam2, 32, 128, 32, wa2, ba2, wb2_, bb2, p["eps2"])
    x2 = _bn_apply(y2, st2, _g2(p["bng2"]), 32)
    y3, st3 = gine(x2, eam3, 32, 128, 32, wa3, ba3, wb3, bb3, p["eps3"])
    x3 = _bn_apply(y3, st3, _g2(p["bng3"]), 32)
    y4, st4 = gine(x3, eam4, 32, 32, 48, wa4, ba4, wb4, bb4, p["eps4"])
    x4 = _bn_apply(y4, st4, _g2(p["bng4"]), 48)

    def edgeconv(xn, Al, F, H, wbot, bias1, wegb, begb, bn_g2, proj):
        D, Ag = _edge_fetch(xn, Al, de_s, se_s, F, H)
        m = _edge_mlp(D, Ag, wbot, bias1, wegb, begb, F, H)
        o = _seg_max(m, de_s, es, H)
        st = _stats2(o, H, relu_in=True)
        return _bn_apply(o, st, bn_g2, H, relu_in=True, proj=proj)

    xe1, A2 = edgeconv(pos_p, A1, 16, 64, eg1b, eg1bias, weg1b, beg1b,
                       _g2(p["bn1"]), eg2t)
    xe2, A3 = edgeconv(xe1, A2, 64, 256, eg2b, eg2bias, weg2b, beg2b,
                       _g2(p["bn2"]), eg3t)
    xe3, A4 = edgeconv(xe2, A3, 256, 256, eg3b, eg3bias, weg3b, beg3b,
                       _g2(p["bn3"]), eg4t)
    xe4 = edgeconv(xe3, A4, 256, 208, eg4b, eg4bias, weg4b, beg4b,
                   _g2(p["bn4"]), None)

    return _pool_final(xe4, x4, batch_p, _g2(p["bn6"]))


# speedup vs baseline: 1.1823x; 1.0733x over previous
import functools

import jax
import jax.numpy as jnp
from jax import lax
from jax.experimental import pallas as pl
from jax.experimental.pallas import tpu as pltpu
from jax.experimental.pallas import tpu_sc as plsc

NN = 10000
EE = 160000
GG = 64
NP = 10240
EP = 163840
NW = 32
EPW = EP // NW
NPW = NP // NW
RB = 512
NB = NP // RB
F32 = jnp.float32
I32 = jnp.int32

_SC_PARAMS = pltpu.CompilerParams(use_tc_tiling_on_sc=False,
                                  needs_layout_passes=False)


def _fs(shape):
    return pl.BlockSpec(shape, lambda i: tuple(0 for _ in shape))


def _rows_mask(i):
    rows = lax.broadcasted_iota(I32, (RB, 1), 0) + i * RB
    return rows < NN


def _dot(a, b, hi=False):
    return jnp.dot(a, b, preferred_element_type=F32,
                   precision=(lax.Precision.HIGHEST if hi
                              else lax.Precision.DEFAULT))


def _embed_nodes(xi_p, pos_p, e1, e2, e3, e4, w1top):
    def body(xi_ref, pos_ref, e1_ref, e2_ref, e3_ref, e4_ref, wt_ref,
             x_ref, a_ref):
        xi = xi_ref[...]
        oh1 = (xi[:, 0:1] == lax.broadcasted_iota(I32, (RB, 16), 1)).astype(F32)
        oh2 = (xi[:, 1:2] == lax.broadcasted_iota(I32, (RB, 8), 1)).astype(F32)
        oh3 = (xi[:, 2:3] == lax.broadcasted_iota(I32, (RB, 8), 1)).astype(F32)
        oh4 = (xi[:, 3:4] == lax.broadcasted_iota(I32, (RB, 8), 1)).astype(F32)
        c1 = _dot(oh1, e1_ref[...], hi=True)
        c2 = (_dot(oh2, e2_ref[...], hi=True)
              + _dot(oh3, e3_ref[...], hi=True)
              + _dot(oh4, e4_ref[...], hi=True))
        c3 = (xi[:, 4:5] - 1).astype(F32)
        x_ref[...] = jnp.concatenate(
            [c1, c2, c3, jnp.zeros((RB, 15), F32)], axis=1)
        a_ref[...] = _dot(pos_ref[...], wt_ref[...])

    return pl.pallas_call(
        body, grid=(NB,),
        in_specs=[pl.BlockSpec((RB, 8), lambda i: (i, 0)),
                  pl.BlockSpec((RB, 16), lambda i: (i, 0)),
                  _fs((16, 32)), _fs((8, 32)), _fs((8, 32)), _fs((8, 32)),
                  _fs((16, 64))],
        out_specs=[pl.BlockSpec((RB, 80), lambda i: (i, 0)),
                   pl.BlockSpec((RB, 64), lambda i: (i, 0))],
        out_shape=[jax.ShapeDtypeStruct((NP, 80), F32),
                   jax.ShapeDtypeStruct((NP, 64), F32)],
    )(xi_p, pos_p, e1, e2, e3, e4, w1top)


def _embed_edges(ai_p, embp, w1, bb1, w2, bb2, w3, bb3, w4, bb4):
    def body(ai_ref, emb_ref, w1_ref, b1_ref, w2_ref, b2_ref, w3_ref, b3_ref,
             w4_ref, b4_ref, o1_ref, o2_ref, o3_ref, o4_ref):
        ai = ai_ref[...]
        oh = (ai[:, 0:1] == lax.broadcasted_iota(I32, (RB, 8), 1)).astype(F32)
        et = _dot(oh, emb_ref[...], hi=True)
        ea = jnp.concatenate([et[:, :15], (ai[:, 1:2] - 1).astype(F32)], axis=1)
        o1_ref[...] = _dot(ea, w1_ref[...]) + b1_ref[...]
        o2_ref[...] = _dot(ea, w2_ref[...]) + b2_ref[...]
        o3_ref[...] = _dot(ea, w3_ref[...]) + b3_ref[...]
        o4_ref[...] = _dot(ea, w4_ref[...]) + b4_ref[...]

    return pl.pallas_call(
        body, grid=(EP // RB,),
        in_specs=[pl.BlockSpec((RB, 8), lambda i: (i, 0)),
                  _fs((8, 16)), _fs((16, 80)), _fs((1, 80)),
                  _fs((16, 32)), _fs((1, 32)), _fs((16, 32)), _fs((1, 32)),
                  _fs((16, 32)), _fs((1, 32))],
        out_specs=[pl.BlockSpec((RB, 80), lambda i: (i, 0)),
                   pl.BlockSpec((RB, 32), lambda i: (i, 0)),
                   pl.BlockSpec((RB, 32), lambda i: (i, 0)),
                   pl.BlockSpec((RB, 32), lambda i: (i, 0))],
        out_shape=[jax.ShapeDtypeStruct((EP, 80), F32),
                   jax.ShapeDtypeStruct((EP, 32), F32),
                   jax.ShapeDtypeStruct((EP, 32), F32),
                   jax.ShapeDtypeStruct((EP, 32), F32)],
    )(ai_p, embp, w1, bb1, w2, bb2, w3, bb3, w4, bb4)


def _gine_mlp(x, parts, wa, ba, wb, bb, eps, F, Fh, Fo):
    def body(x_ref, p_ref, wa_ref, ba_ref, wb_ref, bb_ref, eps_ref, y_ref):
        h = (1.0 + eps_ref[0, 0]) * x_ref[...] + p_ref[0] + p_ref[1]
        t = jnp.maximum(_dot(h, wa_ref[...]) + ba_ref[...], 0.0)
        y_ref[...] = jnp.maximum(_dot(t, wb_ref[...]) + bb_ref[...], 0.0)

    return pl.pallas_call(
        body, grid=(NB,),
        in_specs=[pl.BlockSpec((RB, F), lambda i: (i, 0)),
                  pl.BlockSpec((2, RB, F), lambda i: (0, i, 0)),
                  _fs((F, Fh)), _fs((1, Fh)), _fs((Fh, Fo)), _fs((1, Fo)),
                  _fs((1, 1))],
        out_specs=pl.BlockSpec((RB, Fo), lambda i: (i, 0)),
        out_shape=jax.ShapeDtypeStruct((NP, Fo), F32),
    )(x, parts, wa, ba, wb, bb, eps)


def _stats2(y, F, relu_in=False):
    def body(y_ref, st_ref):
        i = pl.program_id(0)
        yv = y_ref[...]
        if relu_in:
            yv = jnp.maximum(yv, 0.0)
        mask = _rows_mask(i % NB)

        @pl.when(i == 0)
        def _():
            st_ref[...] = jnp.zeros((8, F), F32)

        @pl.when(i < NB)
        def _():
            ym = jnp.where(mask, yv, 0.0)
            st_ref[0:1, :] += jnp.sum(ym, axis=0, keepdims=True)

        @pl.when(i >= NB)
        def _():
            mean = st_ref[0:1, :] / NN
            d = jnp.where(mask, yv - mean, 0.0)
            st_ref[1:2, :] += jnp.sum(d * d, axis=0, keepdims=True)

    return pl.pallas_call(
        body, grid=(2 * NB,),
        in_specs=[pl.BlockSpec((RB, F), lambda i: (i % NB, 0))],
        out_specs=pl.BlockSpec((8, F), lambda i: (0, 0)),
        out_shape=jax.ShapeDtypeStruct((8, F), F32),
    )(y)


def _bn_apply(y, st, g2, F, relu_in=False, proj=None):
    H = proj.shape[1] if proj is not None else 0

    def body(*refs):
        if proj is not None:
            y_ref, st_ref, g_ref, wt_ref, xn_ref, a_ref = refs
        else:
            y_ref, st_ref, g_ref, xn_ref = refs
        mean = st_ref[0:1, :] / NN
        var = st_ref[1:2, :] / NN
        rs = lax.rsqrt(var + 1e-5)
        yv = y_ref[...]
        if relu_in:
            yv = jnp.maximum(yv, 0.0)
        xn = (yv - mean) * rs * g_ref[0:1, :] + g_ref[1:2, :]
        xn_ref[...] = xn
        if proj is not None:
            a_ref[...] = _dot(xn, wt_ref[...])

    in_specs = [pl.BlockSpec((RB, F), lambda i: (i, 0)),
                _fs((8, F)), _fs((2, F))]
    out_specs = [pl.BlockSpec((RB, F), lambda i: (i, 0))]
    out_shape = [jax.ShapeDtypeStruct((NP, F), F32)]
    args = [y, st, g2]
    if proj is not None:
        in_specs += [_fs((F, H))]
        out_specs += [pl.BlockSpec((RB, H), lambda i: (i, 0))]
        out_shape += [jax.ShapeDtypeStruct((NP, H), F32)]
        args += [proj]
    res = pl.pallas_call(body, grid=(NB,), in_specs=in_specs,
                         out_specs=out_specs, out_shape=out_shape)(*args)
    return res if proj is not None else res[0]


def _edge_mlp(D, Ag, wbot, b1, wb, b2, Fd, H):
    def body(d_ref, ag_ref, wbot_ref, b1_ref, wb_ref, b2_ref, m_ref):
        rh = jnp.maximum(
            ag_ref[...] + _dot(d_ref[...], wbot_ref[...]) + b1_ref[...], 0.0)
        m_ref[...] = _dot(rh, wb_ref[...]) + b2_ref[...]

    return pl.pallas_call(
        body, grid=(EP // RB,),
        in_specs=[pl.BlockSpec((RB, Fd), lambda i: (i, 0)),
                  pl.BlockSpec((RB, H), lambda i: (i, 0)),
                  _fs((Fd, H)), _fs((1, H)), _fs((H, H)), _fs((1, H))],
        out_specs=pl.BlockSpec((RB, H), lambda i: (i, 0)),
        out_shape=jax.ShapeDtypeStruct((EP, H), F32),
    )(D, Ag, wbot, b1, wb, b2)


def _pool_final(xe4, x4, batch_p, g2):
    def body(xe_ref, x4_ref, b_ref, g_ref, o_ref, acc):
        i = pl.program_id(0)

        @pl.when(i == 0)
        def _():
            acc[...] = jnp.zeros((GG, 256), F32)

        oh = (b_ref[...] == lax.broadcasted_iota(I32, (RB, GG), 1)).astype(F32)
        xcat = jnp.concatenate([xe_ref[...], x4_ref[...]], axis=1)
        acc[...] += lax.dot_general(oh, xcat, (((0,), (0,)), ((), ())),
                                    preferred_element_type=F32,
                                    precision=lax.Precision.HIGHEST)

        @pl.when(i == NB - 1)
        def _():
            pv = jnp.maximum(acc[...], 0.0)
            m = jnp.sum(pv, axis=0, keepdims=True) / GG
            d = pv - m
            v = jnp.sum(d * d, axis=0, keepdims=True) / GG
            rs = lax.rsqrt(v + 1e-5)
            o_ref[...] = d * rs * g_ref[0:1, :] + g_ref[1:2, :]

    return pl.pallas_call(
        body, grid=(NB,),
        in_specs=[pl.BlockSpec((RB, 208), lambda i: (i, 0)),
                  pl.BlockSpec((RB, 48), lambda i: (i, 0)),
                  pl.BlockSpec((RB, 1), lambda i: (i, 0)),
                  _fs((2, 256))],
        out_specs=pl.BlockSpec((GG, 256), lambda i: (0, 0)),
        out_shape=jax.ShapeDtypeStruct((GG, 256), F32),
        scratch_shapes=[pltpu.VMEM((GG, 256), F32)],
    )(xe4, x4, batch_p, g2)


@functools.cache
def _mesh():
    return plsc.VectorSubcoreMesh(core_axis_name="c", subcore_axis_name="s",
                                  num_cores=2, num_subcores=16)


def _gine_agg(x, eam, srcp, dstp, F):
    C = 128
    NSUB = 16
    ZR = NP // NSUB

    @functools.partial(
        pl.kernel,
        out_type=jax.ShapeDtypeStruct((2, NP, F), F32),
        mesh=_mesh(),
        compiler_params=_SC_PARAMS,
        scratch_types=[
            pltpu.VMEM((2, C), I32),
            pltpu.VMEM((2, C), I32),
            pltpu.VMEM((2, C, F), F32),
            pltpu.VMEM((2, C, F), F32),
            pltpu.VMEM((C, F), F32),
            pltpu.VMEM_SHARED((NP, F), F32),
            pltpu.SemaphoreType.DMA,
            pltpu.SemaphoreType.DMA,
        ])
    def k(x_h, eam_h, src_h, dst_h, out_h, src_v, dst_v, rows_v, eam_v,
          msg_v, acc, sem, sem2):
        c = lax.axis_index("c")
        s = lax.axis_index("s")
        wid = s * 2 + c

        @plsc.parallel_loop(0, C, unroll=8)
        def zr_row(i):
            for j in range(F // 16):
                msg_v[i, pl.ds(j * 16, 16)] = jnp.zeros((16,), F32)

        def zcopy(i, _):
            pltpu.sync_copy(msg_v, acc.at[pl.ds(s * ZR + i * C, C)])
            return _
        lax.fori_loop(0, ZR // C, zcopy, None)
        plsc.subcore_barrier()

        ebase = wid * EPW
        NCH = EPW // C
        sems = (sem, sem2)

        def issue(ci, bi):
            b = ebase + ci * C
            pltpu.sync_copy(src_h.at[pl.ds(b, C)], src_v.at[bi])
            pltpu.sync_copy(dst_h.at[pl.ds(b, C)], dst_v.at[bi])
            pltpu.async_copy(x_h.at[src_v.at[bi]], rows_v.at[bi], sems[bi])
            pltpu.sync_copy(eam_h.at[pl.ds(b, C), :], eam_v.at[bi])

        def finish(ci, bi):
            pltpu.make_async_copy(
                x_h.at[src_v.at[bi]], rows_v.at[bi], sems[bi]).wait()

            @plsc.parallel_loop(0, C, unroll=8)
            def erow(e):
                for j in range(F // 16):
                    sl = pl.ds(j * 16, 16)
                    msg_v[e, sl] = jnp.maximum(
                        rows_v[bi, e, sl] + eam_v[bi, e, sl], 0.0)
            pltpu.sync_copy(msg_v, acc.at[dst_v.at[bi]], add=True)

        issue(0, 0)

        def pair(kk, _):
            i0 = kk * 2
            issue(i0 + 1, 1)
            finish(i0, 0)

            @pl.when(kk < NCH // 2 - 1)
            def _():
                issue(i0 + 2, 0)
            finish(i0 + 1, 1)
            return _
        lax.fori_loop(0, NCH // 2, pair, None)
        plsc.subcore_barrier()

        def wb(i, _):
            r = s * ZR + i * C
            pltpu.sync_copy(acc.at[pl.ds(r, C)], msg_v)
            pltpu.sync_copy(msg_v, out_h.at[c, pl.ds(r, C)])
            return _
        lax.fori_loop(0, ZR // C, wb, None)

    return k(x, eam, srcp, dstp)


def _edge_fetch(x, A, de_s, se_s, F, H):
    C = 64
    NCH = EPW // C

    @functools.partial(
        pl.kernel,
        out_type=(jax.ShapeDtypeStruct((EP, F), F32),
                  jax.ShapeDtypeStruct((EP, H), F32)),
        mesh=_mesh(),
        compiler_params=_SC_PARAMS,
        scratch_types=[
            pltpu.VMEM((2, C), I32),
            pltpu.VMEM((2, C), I32),
            pltpu.VMEM((2, C, F), F32),
            pltpu.VMEM((2, C, F), F32),
            pltpu.VMEM((2, C, H), F32),
            pltpu.SemaphoreType.DMA,
            pltpu.SemaphoreType.DMA,
            pltpu.SemaphoreType.DMA,
            pltpu.SemaphoreType.DMA,
            pltpu.SemaphoreType.DMA,
            pltpu.SemaphoreType.DMA,
        ])
    def k(x_h, a_h, de_h, se_h, d_out, ag_out, di_v, si_v, xs_v, xd_v, ag_v,
          s10, s20, s30, s11, s21, s31):
        c = lax.axis_index("c")
        s = lax.axis_index("s")
        wid = s * 2 + c
        ebase = wid * EPW
        sems = ((s10, s20, s30), (s11, s21, s31))

        def issue(ci, bi):
            b = ebase + ci * C
            pltpu.sync_copy(de_h.at[pl.ds(b, C)], di_v.at[bi])
            pltpu.sync_copy(se_h.at[pl.ds(b, C)], si_v.at[bi])
            pltpu.async_copy(x_h.at[si_v.at[bi]], xs_v.at[bi], sems[bi][0])
            pltpu.async_copy(x_h.at[di_v.at[bi]], xd_v.at[bi], sems[bi][1])
            pltpu.async_copy(a_h.at[di_v.at[bi]], ag_v.at[bi], sems[bi][2])

        def finish(ci, bi):
            pltpu.make_async_copy(
                x_h.at[si_v.at[bi]], xs_v.at[bi], sems[bi][0]).wait()
            pltpu.make_async_copy(
                x_h.at[di_v.at[bi]], xd_v.at[bi], sems[bi][1]).wait()
            pltpu.make_async_copy(
                a_h.at[di_v.at[bi]], ag_v.at[bi], sems[bi][2]).wait()
            b = ebase + ci * C

            @plsc.parallel_loop(0, C, unroll=8)
            def erow(e):
                for j in range(F // 16):
                    sl = pl.ds(j * 16, 16)
                    xs_v[bi, e, sl] = xs_v[bi, e, sl] - xd_v[bi, e, sl]
            pltpu.sync_copy(xs_v.at[bi], d_out.at[pl.ds(b, C), :])
            pltpu.sync_copy(ag_v.at[bi], ag_out.at[pl.ds(b, C), :])

        issue(0, 0)

        def pair(kk, _):
            i0 = kk * 2
            issue(i0 + 1, 1)
            finish(i0, 0)

            @pl.when(kk < NCH // 2 - 1)
            def _():
                issue(i0 + 2, 0)
            finish(i0 + 1, 1)
            return _
        lax.fori_loop(0, NCH // 2, pair, None)

    return k(x, A, de_s, se_s)


def _seg_max(m, dstp, es, H):
    C = 64

    @functools.partial(
        pl.kernel,
        out_type=jax.ShapeDtypeStruct((NP, H), F32),
        mesh=_mesh(),
        compiler_params=_SC_PARAMS,
        scratch_types=[
            pltpu.VMEM((2, C + 16), I32),
            pltpu.VMEM((2, C, H), F32),
            pltpu.VMEM((NPW, H), F32),
            pltpu.VMEM((48,), I32),
            pltpu.SemaphoreType.DMA,
            pltpu.SemaphoreType.DMA,
        ])
    def k(m_h, dst_h, es_h, out_h, di_v, m_v, acc_v, es_v, ms0, ms1):
        c = lax.axis_index("c")
        s = lax.axis_index("s")
        wid = s * 2 + c
        n0 = wid * NPW
        pltpu.sync_copy(es_h, es_v)

        msem = (ms0, ms1)
        e0 = es_v[pl.ds(wid, 16)][0]
        e1 = es_v[pl.ds(wid + 1, 16)][0]
        a0 = jnp.bitwise_and(e0, jnp.int32(-C))
        nch = (e1 - a0 + (C - 1)) // C

        neg = jnp.full((16,), -jnp.inf, F32)

        @plsc.parallel_loop(0, NPW, unroll=8)
        def ib(i):
            for j in range(H // 16):
                acc_v[i, pl.ds(j * 16, 16)] = neg

        def issue(i, bi):
            b = pl.multiple_of(a0 + i * C, C)
            pltpu.sync_copy(dst_h.at[pl.ds(b, C)], di_v.at[bi, pl.ds(0, C)])
            pltpu.async_copy(m_h.at[pl.ds(b, C), :], m_v.at[bi], msem[bi])

        def finish(i, bi):
            b = pl.multiple_of(a0 + i * C, C)
            pltpu.make_async_copy(
                m_h.at[pl.ds(b, C), :], m_v.at[bi], msem[bi]).wait()

            def erow(e, _):
                d = di_v[bi, pl.ds(e, 16)][0]
                ok = jnp.logical_and(d >= n0, d < n0 + NPW)

                @pl.when(ok)
                def _():
                    dl = d - n0
                    for j in range(H // 16):
                        sl = pl.ds(j * 16, 16)
                        acc_v[dl, sl] = jnp.maximum(acc_v[dl, sl],
                                                    m_v[bi, e, sl])
                return _
            lax.fori_loop(0, C, erow, None)

        @pl.when(nch > 0)
        def _():
            issue(0, 0)

            def pair(kk, _):
                i0 = kk * 2

                @pl.when(i0 + 1 < nch)
                def _():
                    issue(i0 + 1, 1)
                finish(i0, 0)

                @pl.when(i0 + 2 < nch)
                def _():
                    issue(i0 + 2, 0)

                @pl.when(i0 + 1 < nch)
                def _():
                    finish(i0 + 1, 1)
                return _
            lax.fori_loop(0, (nch + 1) // 2, pair, None)
        pltpu.sync_copy(acc_v, out_h.at[pl.ds(n0, NPW)])

    return k(m, dstp, es)


def _pad_rows(a, rows, val=0):
    pad = jnp.full((rows - a.shape[0],) + a.shape[1:], val, a.dtype)
    return jnp.concatenate([a, pad], axis=0)


def _w2(l, fin_pad=None, fout_pad=None):
    w, b = l["w"].astype(F32), l["b"].astype(F32)
    if fin_pad is not None and w.shape[0] < fin_pad:
        w = jnp.concatenate(
            [w, jnp.zeros((fin_pad - w.shape[0], w.shape[1]), F32)], axis=0)
    if fout_pad is not None and w.shape[1] < fout_pad:
        w = jnp.concatenate(
            [w, jnp.zeros((w.shape[0], fout_pad - w.shape[1]), F32)], axis=1)
        b = jnp.concatenate([b, jnp.zeros((fout_pad - b.shape[0],), F32)])
    return w, b[None, :]


def _g2(bn):
    return jnp.stack([bn["g"].astype(F32), bn["b"].astype(F32)], axis=0)


def _ec_split(l, F, fin_pad=None):
    w = l["w"].astype(F32)
    wtop, wbot = w[:F], w[F:]
    if fin_pad is not None and F < fin_pad:
        z = jnp.zeros((fin_pad - F, w.shape[1]), F32)
        wtop = jnp.concatenate([wtop, z], axis=0)
        wbot = jnp.concatenate([wbot, z], axis=0)
    return wtop, wbot, l["b"].astype(F32)[None, :]


def kernel(params, pos, x_int, edge_index, edge_attr_int, batch, edge_index_e):
    p = params

    xi_p = _pad_rows(x_int.astype(I32), NP)
    pos_p = _pad_rows(
        jnp.concatenate([pos.astype(F32), jnp.zeros((NN, 13), F32)], axis=1),
        NP)
    ai_p = _pad_rows(
        jnp.concatenate([edge_attr_int.astype(I32),
                         jnp.zeros((EE, 1), I32)], axis=1), EP)
    src_p = _pad_rows(edge_index[0].astype(I32), EP, 0)
    dst_p = _pad_rows(edge_index[1].astype(I32), EP, NN)
    batch_p = _pad_rows(batch.astype(I32)[:, None], NP, GG)

    de = edge_index_e[1].astype(I32)
    se = edge_index_e[0].astype(I32)
    order = jnp.argsort(de)
    de_s = _pad_rows(de[order], EP, NN)
    se_s = _pad_rows(se[order], EP, 0)
    bounds = jnp.arange(0, NP + NPW, NPW, dtype=I32)
    es = jnp.searchsorted(de_s, bounds, side="left").astype(I32)
    es = jnp.concatenate([es, jnp.full((48 - 33,), EP, I32)])

    emb1 = p["emb1"].astype(F32)
    emb2 = _pad_rows(p["emb2"].astype(F32), 8)
    emb3 = _pad_rows(p["emb3"].astype(F32), 8)
    emb4 = _pad_rows(p["emb4"].astype(F32), 8)
    embp = jnp.concatenate(
        [_pad_rows(p["edge_emb"].astype(F32), 8), jnp.zeros((8, 1), F32)],
        axis=1)
    we1, be1 = _w2(p["lin_e1"], fout_pad=80)
    we2, be2 = _w2(p["lin_e2"])
    we3, be3 = _w2(p["lin_e3"])
    we4, be4 = _w2(p["lin_e4"])
    wa1, ba1 = _w2(p["nn1a"], fin_pad=80, fout_pad=80)
    wb1, bb1 = _w2(p["nn1b"], fin_pad=80)
    wa2, ba2 = _w2(p["nn2a"])
    wb2_, bb2 = _w2(p["nn2b"])
    wa3, ba3 = _w2(p["nn3a"])
    wb3, bb3 = _w2(p["nn3b"])
    wa4, ba4 = _w2(p["nn4a"])
    wb4, bb4 = _w2(p["nn4b"])
    eg1t, eg1b, eg1bias = _ec_split(p["eg1a"], 3, fin_pad=16)
    eg2t, eg2b, eg2bias = _ec_split(p["eg2a"], 64)
    eg3t, eg3b, eg3bias = _ec_split(p["eg3a"], 256)
    eg4t, eg4b, eg4bias = _ec_split(p["eg4a"], 256)
    weg1b, beg1b = _w2(p["eg1b"])
    weg2b, beg2b = _w2(p["eg2b"])
    weg3b, beg3b = _w2(p["eg3b"])
    weg4b, beg4b = _w2(p["eg4b"])

    x0, A1 = _embed_nodes(xi_p, pos_p, emb1, emb2, emb3, emb4, eg1t)
    eam1, eam2, eam3, eam4 = _embed_edges(
        ai_p, embp, we1, be1, we2, be2, we3, be3, we4, be4)

    def gine(xl, eaml, F, Fh, Fo, wa, ba, wb, bb, eps):
        parts = _gine_agg(xl, eaml, src_p, dst_p, F)
        y = _gine_mlp(xl, parts, wa, ba, wb, bb,
                      eps.astype(F32).reshape(1, 1), F, Fh, Fo)
        return y, _stats2(y, Fo)

    y1, st1 = gine(x0, eam1, 80, 80, 32, wa1, ba1, wb1, bb1, p["eps1"])
    x1 = _bn_apply(y1, st1, _g2(p["bng1"]), 32)
    y2, st2 = gine(x1, eam2, 32, 128, 32, wa2, ba2, wb2_, bb2, p["eps2"])
    x2 = _bn_apply(y2, st2, _g2(p["bng2"]), 32)
    y3, st3 = gine(x2, eam3, 32, 128, 32, wa3, ba3, wb3, bb3, p["eps3"])
    x3 = _bn_apply(y3, st3, _g2(p["bng3"]), 32)
    y4, st4 = gine(x3, eam4, 32, 32, 48, wa4, ba4, wb4, bb4, p["eps4"])
    x4 = _bn_apply(y4, st4, _g2(p["bng4"]), 48)

    def edgeconv(xn, Al, F, H, wbot, bias1, wegb, begb, bn_g2, proj):
        D, Ag = _edge_fetch(xn, Al, de_s, se_s, F, H)
        m = _edge_mlp(D, Ag, wbot, bias1, wegb, begb, F, H)
        o = _seg_max(m, de_s, es, H)
        st = _stats2(o, H, relu_in=True)
        return _bn_apply(o, st, bn_g2, H, relu_in=True, proj=proj)

    xe1, A2 = edgeconv(pos_p, A1, 16, 64, eg1b, eg1bias, weg1b, beg1b,
                       _g2(p["bn1"]), eg2t)
    xe2, A3 = edgeconv(xe1, A2, 64, 256, eg2b, eg2bias, weg2b, beg2b,
                       _g2(p["bn2"]), eg3t)
    xe3, A4 = edgeconv(xe2, A3, 256, 256, eg3b, eg3bias, weg3b, beg3b,
                       _g2(p["bn3"]), eg4t)
    xe4 = edgeconv(xe3, A4, 256, 208, eg4b, eg4bias, weg4b, beg4b,
                   _g2(p["bn4"]), None)

    return _pool_final(xe4, x4, batch_p, _g2(p["bn6"]))
